# baseline (speedup 1.0000x reference)
import functools

import jax
import jax.numpy as jnp
from jax import lax
from jax.experimental import pallas as pl
from jax.experimental.pallas import tpu as pltpu, tpu_sc as plsc

N = 50000
E = 800000
HID = 64
FEAT = 28

N_PAD = 51200
E_PAD = 819200

NC = 2
NS = 16
LANES = 16

B_POS = 3200
B_CONV = 128
IB_CONV = 16
B_PAIR = 512
IB_PAIR = 10

BE = 1024
BN = 512


def _pos_sumsq_body(px_ref, py_ref, pz_ref, src_ref, dst_ref, out_ref,
                    tab_a, tab_b, sbuf, ibuf_s, ibuf_d):
    wid = lax.axis_index("s") * NC + lax.axis_index("c")
    chunk = E_PAD // (NC * NS)
    nblk = chunk // B_POS
    base = wid * chunk

    pltpu.sync_copy(px_ref, tab_a)
    pltpu.sync_copy(py_ref, tab_b)

    def blk1(b, _):
        e0 = base + b * B_POS
        pltpu.sync_copy(src_ref.at[pl.ds(e0, B_POS)], ibuf_s)
        pltpu.sync_copy(dst_ref.at[pl.ds(e0, B_POS)], ibuf_d)

        def inner(j, _):
            sl = pl.ds(j * LANES, LANES)
            isv = ibuf_s[sl]
            idv = ibuf_d[sl]
            dx = plsc.load_gather(tab_a, [idv]) - plsc.load_gather(tab_a, [isv])
            dy = plsc.load_gather(tab_b, [idv]) - plsc.load_gather(tab_b, [isv])
            sbuf[sl] = dx * dx + dy * dy
            return 0

        lax.fori_loop(0, B_POS // LANES, inner, 0)
        pltpu.sync_copy(sbuf, out_ref.at[pl.ds(e0, B_POS)])
        return 0

    lax.fori_loop(0, nblk, blk1, 0)

    pltpu.sync_copy(pz_ref, tab_a)

    def blk2(b, _):
        e0 = base + b * B_POS
        pltpu.sync_copy(src_ref.at[pl.ds(e0, B_POS)], ibuf_s)
        pltpu.sync_copy(dst_ref.at[pl.ds(e0, B_POS)], ibuf_d)
        pltpu.sync_copy(out_ref.at[pl.ds(e0, B_POS)], sbuf)

        def inner(j, _):
            sl = pl.ds(j * LANES, LANES)
            dz = (plsc.load_gather(tab_a, [ibuf_d[sl]])
                  - plsc.load_gather(tab_a, [ibuf_s[sl]]))
            sbuf[sl] = sbuf[sl] + dz * dz
            return 0

        lax.fori_loop(0, B_POS // LANES, inner, 0)
        pltpu.sync_copy(sbuf, out_ref.at[pl.ds(e0, B_POS)])
        return 0

    lax.fori_loop(0, nblk, blk2, 0)


def _pos_sumsq(px, py, pz, src, dst):
    mesh = plsc.VectorSubcoreMesh(core_axis_name="c", subcore_axis_name="s")
    return pl.kernel(
        _pos_sumsq_body,
        out_type=jax.ShapeDtypeStruct((E_PAD,), jnp.float32),
        mesh=mesh,
        scratch_types=[
            pltpu.VMEM((N_PAD,), jnp.float32),
            pltpu.VMEM((N_PAD,), jnp.float32),
            pltpu.VMEM((B_POS,), jnp.float32),
            pltpu.VMEM((B_POS,), jnp.int32),
            pltpu.VMEM((B_POS,), jnp.int32),
        ],
        compiler_params=pltpu.CompilerParams(needs_layout_passes=False),
    )(px, py, pz, src, dst)


def _node_embed_body(at_ref, rf_ref, pf_ref, aemb_ref, wf_ref, wm_ref,
                     h_ref, g_ref):
    ids = at_ref[:, 0]
    oh = (ids[:, None] == lax.broadcasted_iota(jnp.int32, (BN, 100), 1))
    a_emb = jnp.dot(oh.astype(jnp.float32), aemb_ref[...],
                    preferred_element_type=jnp.float32,
                    precision=lax.Precision.HIGHEST)
    af_r = jnp.dot(rf_ref[...], wf_ref[...], preferred_element_type=jnp.float32,
                   precision=lax.Precision.HIGHEST)
    af_p = jnp.dot(pf_ref[...], wf_ref[...], preferred_element_type=jnp.float32,
                   precision=lax.Precision.HIGHEST)
    z = jnp.concatenate([a_emb + af_r, af_p - af_r], axis=-1)
    h_ref[...] = z
    g = jnp.dot(z, wm_ref[...], preferred_element_type=jnp.float32,
                precision=lax.Precision.HIGHEST)
    g_ref[0] = g[:, :32]
    g_ref[1] = g[:, 32:]


def _node_embed(at, rf, pf, atom_emb, w_feat, wm0):
    grid = N_PAD // BN
    return pl.pallas_call(
        _node_embed_body,
        grid=(grid,),
        in_specs=[
            pl.BlockSpec((BN, 1), lambda i: (i, 0)),
            pl.BlockSpec((BN, FEAT), lambda i: (i, 0)),
            pl.BlockSpec((BN, FEAT), lambda i: (i, 0)),
            pl.BlockSpec((100, 32), lambda i: (0, 0)),
            pl.BlockSpec((FEAT, 32), lambda i: (0, 0)),
            pl.BlockSpec((HID, HID), lambda i: (0, 0)),
        ],
        out_specs=[
            pl.BlockSpec((BN, HID), lambda i: (i, 0)),
            pl.BlockSpec((2, BN, 32), lambda i: (0, i, 0)),
        ],
        out_shape=[
            jax.ShapeDtypeStruct((N_PAD, HID), jnp.float32),
            jax.ShapeDtypeStruct((2, N_PAD, 32), jnp.float32),
        ],
    )(at, rf, pf, atom_emb, w_feat, wm0)


def _edge_base_body(ss_ref, bt_ref, we1_ref, be1_ref, we2_ref, be2_ref,
                    bemb_ref, wc1_ref, bc1_ref, wc2_ref, bc2_ref, wep_ref,
                    ea_ref, ep0_ref):
    pid = pl.program_id(0)
    el = jnp.sqrt(ss_ref[...] + 1e-12)
    hd = jax.nn.relu(el * we1_ref[0][None, :] + be1_ref[0][None, :])
    hd = jnp.dot(hd, we2_ref[...], preferred_element_type=jnp.float32) \
        + be2_ref[0][None, :]
    bt = bt_ref[...]
    bemb = ((bt == 0) * bemb_ref[0][None, :] + (bt == 1) * bemb_ref[1][None, :]
            + (bt == 2) * bemb_ref[2][None, :] + (bt == 3) * bemb_ref[3][None, :])
    attr = hd * bemb
    ea = jax.nn.relu(jnp.dot(attr, wc1_ref[...],
                             preferred_element_type=jnp.float32)
                     + bc1_ref[0][None, :])
    ea = jnp.dot(ea, wc2_ref[...], preferred_element_type=jnp.float32) \
        + bc2_ref[0][None, :]
    eidx = pid * BE + lax.broadcasted_iota(jnp.int32, (BE, 1), 0)
    mask = (eidx < E).astype(jnp.float32)
    ea_ref[...] = ea
    ep = jnp.dot(ea, wep_ref[...], preferred_element_type=jnp.float32) * mask
    ep0_ref[0] = ep[:, 0:32]
    ep0_ref[1] = ep[:, 32:64]


def _edge_base(sumsq, bt, p):
    grid = E_PAD // BE
    wvec = lambda shp: pl.BlockSpec(shp, lambda i: (0, 0))
    wc1s = p["Wc1"][:HID] + p["Wc1"][HID:]
    return pl.pallas_call(
        _edge_base_body,
        grid=(grid,),
        in_specs=[
            pl.BlockSpec((BE, 1), lambda i: (i, 0)),
            pl.BlockSpec((BE, 1), lambda i: (i, 0)),
            wvec((1, HID)), wvec((1, HID)),
            wvec((HID, HID)), wvec((1, HID)),
            wvec((4, HID)),
            wvec((HID, HID)), wvec((1, HID)),
            wvec((HID, HID)), wvec((1, HID)),
            wvec((HID, HID)),
        ],
        out_specs=[
            pl.BlockSpec((BE, HID), lambda i: (i, 0)),
            pl.BlockSpec((2, BE, 32), lambda i: (0, i, 0)),
        ],
        out_shape=[
            jax.ShapeDtypeStruct((E_PAD, HID), jnp.float32),
            jax.ShapeDtypeStruct((2, E_PAD, 32), jnp.float32),
        ],
    )(sumsq, bt, p["We1"], p["be1"].reshape(1, HID), p["We2"],
      p["be2"].reshape(1, HID), p["bond_emb"][:4], wc1s,
      p["bc1"].reshape(1, HID), p["Wc2"], p["bc2"].reshape(1, HID), p["Wep0"])


def _edge_ep12_body(ea_ref, wep_ref, ep1_ref, ep2_ref):
    pid = pl.program_id(0)
    eidx = pid * BE + lax.broadcasted_iota(jnp.int32, (BE, 1), 0)
    mask = (eidx < E).astype(jnp.float32)
    ep = jnp.dot(ea_ref[...], wep_ref[...],
                 preferred_element_type=jnp.float32) * mask
    ep1_ref[0] = ep[:, 0:32]
    ep1_ref[1] = ep[:, 32:64]
    ep2_ref[0] = ep[:, 64:96]
    ep2_ref[1] = ep[:, 96:128]


def _edge_ep12(ea, p):
    grid = E_PAD // BE
    ep_spec = pl.BlockSpec((2, BE, 32), lambda i: (0, i, 0))
    ep_shape = jax.ShapeDtypeStruct((2, E_PAD, 32), jnp.float32)
    wep12 = jnp.concatenate([p["Wep1"], p["Wep2"]], axis=1)
    return pl.pallas_call(
        _edge_ep12_body,
        grid=(grid,),
        in_specs=[
            pl.BlockSpec((BE, HID), lambda i: (i, 0)),
            pl.BlockSpec((HID, 2 * HID), lambda i: (0, 0)),
        ],
        out_specs=[ep_spec, ep_spec],
        out_shape=[ep_shape, ep_shape],
    )(ea, wep12)


def _conv_body(g_ref, ep_ref, src_ref, dst_ref, agg_ref,
               accum, gbuf, ebuf, isbuf, idbuf,
               sem_g0, sem_g1, sem_e0, sem_e1, sem_s0, sem_s1):
    c = lax.axis_index("c")
    s_id = lax.axis_index("s")
    rows_per_tile = N_PAD // NS
    chunk = E_PAD // NS
    sbsz = IB_CONV * B_CONV
    nsb = chunk // sbsz
    sem_g = (sem_g0, sem_g1)
    sem_e = (sem_e0, sem_e1)
    sem_s = (sem_s0, sem_s1)

    def zrow(j, _):
        gbuf[0, j, pl.ds(0, LANES)] = jnp.zeros((LANES,), jnp.float32)
        gbuf[0, j, pl.ds(LANES, LANES)] = jnp.zeros((LANES,), jnp.float32)
        return 0

    lax.fori_loop(0, B_CONV, zrow, 0)
    r0 = s_id * rows_per_tile
    nfull = rows_per_tile // B_CONV
    rem = rows_per_tile - nfull * B_CONV

    def zcp(k, _):
        pltpu.sync_copy(gbuf.at[0], accum.at[pl.ds(r0 + k * B_CONV, B_CONV)])
        return 0

    lax.fori_loop(0, nfull, zcp, 0)
    if rem:
        pltpu.sync_copy(gbuf.at[0, pl.ds(0, rem)],
                        accum.at[pl.ds(r0 + nfull * B_CONV, rem)])
    plsc.subcore_barrier()

    def sblock(sb, _):
        row0 = s_id * (chunk // B_CONV) + sb * IB_CONV
        e_base = s_id * chunk + sb * sbsz
        pltpu.sync_copy(src_ref.at[pl.ds(row0, IB_CONV)], isbuf)
        pltpu.sync_copy(dst_ref.at[pl.ds(row0, IB_CONV)], idbuf)

        def issue(k):
            buf = k % 2
            pltpu.async_copy(g_ref.at[c].at[isbuf.at[k]], gbuf.at[buf],
                             sem_g[buf])
            pltpu.async_copy(
                ep_ref.at[c, pl.ds(e_base + k * B_CONV, B_CONV)],
                ebuf.at[buf], sem_e[buf])

        def wait_in(k):
            buf = k % 2
            pltpu.make_async_copy(g_ref.at[c].at[isbuf.at[k]], gbuf.at[buf],
                                  sem_g[buf]).wait()
            pltpu.make_async_copy(
                ep_ref.at[c, pl.ds(e_base + k * B_CONV, B_CONV)],
                ebuf.at[buf], sem_e[buf]).wait()

        def mul(k):
            buf = k % 2

            def body(j, _):
                lo = pl.ds(0, LANES)
                hi = pl.ds(LANES, LANES)
                gbuf[buf, j, lo] = gbuf[buf, j, lo] * ebuf[buf, j, lo]
                gbuf[buf, j, hi] = gbuf[buf, j, hi] * ebuf[buf, j, hi]
                return 0

            lax.fori_loop(0, B_CONV, body, 0)

        def scatter(k):
            buf = k % 2
            pltpu.async_copy(gbuf.at[buf], accum.at[idbuf.at[k]], sem_s[buf],
                             add=True)

        def wait_scatter(k):
            buf = k % 2
            pltpu.make_async_copy(gbuf.at[buf], accum.at[idbuf.at[k]],
                                  sem_s[buf]).wait()

        issue(0)
        for k in range(IB_CONV):
            wait_in(k)
            if k >= 1:
                wait_scatter(k - 1)
            if k + 1 < IB_CONV:
                issue(k + 1)
            mul(k)
            scatter(k)
        wait_scatter(IB_CONV - 1)
        return 0

    lax.fori_loop(0, nsb, sblock, 0)
    plsc.subcore_barrier()
    pltpu.sync_copy(accum.at[pl.ds(r0, rows_per_tile)],
                    agg_ref.at[c, pl.ds(r0, rows_per_tile)])


def _conv_pass(g, ep, src2, dst2):
    mesh = plsc.VectorSubcoreMesh(core_axis_name="c", subcore_axis_name="s")
    return pl.kernel(
        _conv_body,
        out_type=jax.ShapeDtypeStruct((2, N_PAD, 32), jnp.float32),
        mesh=mesh,
        scratch_types=[
            pltpu.VMEM_SHARED((N_PAD, 32), jnp.float32),
            pltpu.VMEM((2, B_CONV, 32), jnp.float32),
            pltpu.VMEM((2, B_CONV, 32), jnp.float32),
            pltpu.VMEM((IB_CONV, B_CONV), jnp.int32),
            pltpu.VMEM((IB_CONV, B_CONV), jnp.int32),
            pltpu.SemaphoreType.DMA, pltpu.SemaphoreType.DMA,
            pltpu.SemaphoreType.DMA, pltpu.SemaphoreType.DMA,
            pltpu.SemaphoreType.DMA, pltpu.SemaphoreType.DMA,
        ],
        compiler_params=pltpu.CompilerParams(
            needs_layout_passes=False, use_tc_tiling_on_sc=False),
    )(g, ep, src2, dst2)


def _node_update_body(h_ref, agg_ref, wu_ref, bu_ref, wn_ref, hn_ref, g_ref):
    aggc = jnp.concatenate([agg_ref[0], agg_ref[1]], axis=-1)
    hn = h_ref[...] + jax.nn.relu(
        jnp.dot(aggc, wu_ref[...], preferred_element_type=jnp.float32, precision=lax.Precision.HIGHEST)
        + bu_ref[0][None, :])
    hn_ref[...] = hn
    g = jnp.dot(hn, wn_ref[...], preferred_element_type=jnp.float32, precision=lax.Precision.HIGHEST)
    g_ref[0] = g[:, :32]
    g_ref[1] = g[:, 32:]


def _node_update(h, agg, wu, bu, wnext):
    grid = N_PAD // BN
    return pl.pallas_call(
        _node_update_body,
        grid=(grid,),
        in_specs=[
            pl.BlockSpec((BN, HID), lambda i: (i, 0)),
            pl.BlockSpec((2, BN, 32), lambda i: (0, i, 0)),
            pl.BlockSpec((HID, HID), lambda i: (0, 0)),
            pl.BlockSpec((1, HID), lambda i: (0, 0)),
            pl.BlockSpec((HID, HID), lambda i: (0, 0)),
        ],
        out_specs=[
            pl.BlockSpec((BN, HID), lambda i: (i, 0)),
            pl.BlockSpec((2, BN, 32), lambda i: (0, i, 0)),
        ],
        out_shape=[
            jax.ShapeDtypeStruct((N_PAD, HID), jnp.float32),
            jax.ShapeDtypeStruct((2, N_PAD, 32), jnp.float32),
        ],
    )(h, agg, wu, bu.reshape(1, HID), wnext)


def _pair_body(h_ref, src_ref, dst_ref, prod_ref,
               sbuf, dbuf, obuf, isbuf, idbuf,
               sem_a0, sem_a1, sem_b0, sem_b1, sem_w0, sem_w1):
    c = lax.axis_index("c")
    s_id = lax.axis_index("s")
    chunk = E_PAD // NS
    sbsz = IB_PAIR * B_PAIR
    nsb = chunk // sbsz
    sem_a = (sem_a0, sem_a1)
    sem_b = (sem_b0, sem_b1)
    sem_w = (sem_w0, sem_w1)

    def sblock(sb, _):
        row0 = s_id * (chunk // B_PAIR) + sb * IB_PAIR
        e_base = s_id * chunk + sb * sbsz
        pltpu.sync_copy(src_ref.at[pl.ds(row0, IB_PAIR)], isbuf)
        pltpu.sync_copy(dst_ref.at[pl.ds(row0, IB_PAIR)], idbuf)

        def issue(k):
            buf = k % 2
            pltpu.async_copy(h_ref.at[c].at[isbuf.at[k]], sbuf.at[buf],
                             sem_a[buf])
            pltpu.async_copy(h_ref.at[c].at[idbuf.at[k]], dbuf.at[buf],
                             sem_b[buf])

        def wait_in(k):
            buf = k % 2
            pltpu.make_async_copy(h_ref.at[c].at[isbuf.at[k]], sbuf.at[buf],
                                  sem_a[buf]).wait()
            pltpu.make_async_copy(h_ref.at[c].at[idbuf.at[k]], dbuf.at[buf],
                                  sem_b[buf]).wait()

        def mul(k):
            buf = k % 2

            def body(j, _):
                lo = pl.ds(0, LANES)
                hi = pl.ds(LANES, LANES)
                obuf[buf, j, lo] = sbuf[buf, j, lo] * dbuf[buf, j, lo]
                obuf[buf, j, hi] = sbuf[buf, j, hi] * dbuf[buf, j, hi]
                return 0

            lax.fori_loop(0, B_PAIR, body, 0)

        def wr(k):
            buf = k % 2
            pltpu.async_copy(
                obuf.at[buf],
                prod_ref.at[c, pl.ds(e_base + k * B_PAIR, B_PAIR)],
                sem_w[buf])

        def wait_wr(k):
            buf = k % 2
            pltpu.make_async_copy(
                obuf.at[buf],
                prod_ref.at[c, pl.ds(e_base + k * B_PAIR, B_PAIR)],
                sem_w[buf]).wait()

        issue(0)
        for k in range(IB_PAIR):
            wait_in(k)
            if k + 1 < IB_PAIR:
                issue(k + 1)
            if k >= 2:
                wait_wr(k - 2)
            mul(k)
            wr(k)
        wait_wr(IB_PAIR - 2)
        wait_wr(IB_PAIR - 1)
        return 0

    lax.fori_loop(0, nsb, sblock, 0)


def _pair_pass(h_split, src2, dst2):
    mesh = plsc.VectorSubcoreMesh(core_axis_name="c", subcore_axis_name="s")
    return pl.kernel(
        _pair_body,
        out_type=jax.ShapeDtypeStruct((2, E_PAD, 32), jnp.float32),
        mesh=mesh,
        scratch_types=[
            pltpu.VMEM((2, B_PAIR, 32), jnp.float32),
            pltpu.VMEM((2, B_PAIR, 32), jnp.float32),
            pltpu.VMEM((2, B_PAIR, 32), jnp.float32),
            pltpu.VMEM((IB_PAIR, B_PAIR), jnp.int32),
            pltpu.VMEM((IB_PAIR, B_PAIR), jnp.int32),
            pltpu.SemaphoreType.DMA, pltpu.SemaphoreType.DMA,
            pltpu.SemaphoreType.DMA, pltpu.SemaphoreType.DMA,
            pltpu.SemaphoreType.DMA, pltpu.SemaphoreType.DMA,
        ],
        compiler_params=pltpu.CompilerParams(
            needs_layout_passes=False, use_tc_tiling_on_sc=False),
    )(h_split, src2, dst2)


BE_H = 800


def _head_body(prod_ref, ea_ref, ss_ref, wo1_ref, bo1_ref, wo2_ref, bo2_ref,
               wo3_ref, bo3_ref, out_ref, el_ref):
    el_ref[...] = jnp.sqrt(ss_ref[...] + 1e-12)
    hh = jnp.concatenate([prod_ref[0], prod_ref[1], ea_ref[...]], axis=-1)
    o = jax.nn.relu(jnp.dot(hh, wo1_ref[...],
                            preferred_element_type=jnp.float32)
                    + bo1_ref[0][None, :])
    o = jax.nn.relu(jnp.dot(o, wo2_ref[...],
                            preferred_element_type=jnp.float32)
                    + bo2_ref[0][None, :])
    out_ref[...] = jnp.dot(o, wo3_ref[...],
                           preferred_element_type=jnp.float32) \
        + bo3_ref[0][None, :]


def _head(prod, ea, sumsq, p):
    grid = E // BE_H
    wvec = lambda shp: pl.BlockSpec(shp, lambda i: (0, 0))
    return pl.pallas_call(
        _head_body,
        grid=(grid,),
        in_specs=[
            pl.BlockSpec((2, BE_H, 32), lambda i: (0, i, 0)),
            pl.BlockSpec((BE_H, HID), lambda i: (i, 0)),
            pl.BlockSpec((BE_H, 1), lambda i: (i, 0)),
            wvec((2 * HID, HID)), wvec((1, HID)),
            wvec((HID, 32)), wvec((1, 32)),
            wvec((32, 3)), wvec((1, 3)),
        ],
        out_specs=[
            pl.BlockSpec((BE_H, 3), lambda i: (i, 0)),
            pl.BlockSpec((BE_H, 1), lambda i: (i, 0)),
        ],
        out_shape=[
            jax.ShapeDtypeStruct((E, 3), jnp.float32),
            jax.ShapeDtypeStruct((E, 1), jnp.float32),
        ],
    )(prod, ea, sumsq, p["Wo1"], p["bo1"].reshape(1, HID), p["Wo2"],
      p["bo2"].reshape(1, 32), p["Wo3"], p["bo3"].reshape(1, 3))


def kernel(atom_type, r_feat, p_feat, rtsp, pos_N_3, bond_index, bond_type,
           batch, time_step, params):
    p = params
    at = jnp.pad(atom_type.astype(jnp.int32), (0, N_PAD - N)).reshape(N_PAD, 1)
    rf = jnp.pad(r_feat, ((0, N_PAD - N), (0, 0)))
    pf = jnp.pad(p_feat, ((0, N_PAD - N), (0, 0)))
    pos_t = jnp.pad(pos_N_3, ((0, N_PAD - N), (0, 0))).T
    px, py, pz = pos_t[0], pos_t[1], pos_t[2]
    src = jnp.pad(bond_index[0].astype(jnp.int32), (0, E_PAD - E))
    dst = jnp.pad(bond_index[1].astype(jnp.int32), (0, E_PAD - E))
    bt = jnp.pad(bond_type.astype(jnp.int32), (0, E_PAD - E)).reshape(E_PAD, 1)

    src_c = src.reshape(E_PAD // B_CONV, B_CONV)
    dst_c = dst.reshape(E_PAD // B_CONV, B_CONV)
    src_p = src.reshape(E_PAD // B_PAIR, B_PAIR)
    dst_p = dst.reshape(E_PAD // B_PAIR, B_PAIR)

    h, g = _node_embed(at, rf, pf, p["atom_emb"], p["W_feat"], p["Wm0"])
    sumsq = _pos_sumsq(px, py, pz, src, dst).reshape(E_PAD, 1)
    ea, ep0 = _edge_base(sumsq, bt, p)
    ep1, ep2 = _edge_ep12(ea, p)

    eye = jnp.eye(HID, dtype=jnp.float32)
    for i, ep in enumerate((ep0, ep1, ep2)):
        agg = _conv_pass(g, ep, src_c, dst_c)
        wnext = p["Wm%d" % (i + 1)] if i < 2 else eye
        h, g = _node_update(h, agg, p["Wu%d" % i], p["bu%d" % i], wnext)

    prod = _pair_pass(g, src_p, dst_p)
    edge_inv, el = _head(prod, ea, sumsq, p)

    return edge_inv, bond_index, el


# speedup vs baseline: 1.0172x; 1.0172x over previous
import functools

import jax
import jax.numpy as jnp
from jax import lax
from jax.experimental import pallas as pl
from jax.experimental.pallas import tpu as pltpu, tpu_sc as plsc

N = 50000
E = 800000
HID = 64
FEAT = 28

N_PAD = 50176
E_PAD = 819200

NC = 2
NS = 16
LANES = 16

B_POS = 3200
B_CONV = 128
IB_CONV = 16
B_PAIR = 512
IB_PAIR = 10

BE = 1024
BN = 512


def _pos_sumsq_body(px_ref, py_ref, pz_ref, src_ref, dst_ref, out_ref,
                    tab_a, tab_b, sbuf, ibuf_s, ibuf_d):
    wid = lax.axis_index("s") * NC + lax.axis_index("c")
    chunk = E_PAD // (NC * NS)
    nblk = chunk // B_POS
    base = wid * chunk

    pltpu.sync_copy(px_ref, tab_a)
    pltpu.sync_copy(py_ref, tab_b)

    def blk1(b, _):
        e0 = base + b * B_POS
        pltpu.sync_copy(src_ref.at[pl.ds(e0, B_POS)], ibuf_s)
        pltpu.sync_copy(dst_ref.at[pl.ds(e0, B_POS)], ibuf_d)

        def inner(j, _):
            sl = pl.ds(j * LANES, LANES)
            isv = ibuf_s[sl]
            idv = ibuf_d[sl]
            dx = plsc.load_gather(tab_a, [idv]) - plsc.load_gather(tab_a, [isv])
            dy = plsc.load_gather(tab_b, [idv]) - plsc.load_gather(tab_b, [isv])
            sbuf[sl] = dx * dx + dy * dy
            return 0

        lax.fori_loop(0, B_POS // LANES, inner, 0)
        pltpu.sync_copy(sbuf, out_ref.at[pl.ds(e0, B_POS)])
        return 0

    lax.fori_loop(0, nblk, blk1, 0)

    pltpu.sync_copy(pz_ref, tab_a)

    def blk2(b, _):
        e0 = base + b * B_POS
        pltpu.sync_copy(src_ref.at[pl.ds(e0, B_POS)], ibuf_s)
        pltpu.sync_copy(dst_ref.at[pl.ds(e0, B_POS)], ibuf_d)
        pltpu.sync_copy(out_ref.at[pl.ds(e0, B_POS)], sbuf)

        def inner(j, _):
            sl = pl.ds(j * LANES, LANES)
            dz = (plsc.load_gather(tab_a, [ibuf_d[sl]])
                  - plsc.load_gather(tab_a, [ibuf_s[sl]]))
            sbuf[sl] = sbuf[sl] + dz * dz
            return 0

        lax.fori_loop(0, B_POS // LANES, inner, 0)
        pltpu.sync_copy(sbuf, out_ref.at[pl.ds(e0, B_POS)])
        return 0

    lax.fori_loop(0, nblk, blk2, 0)


def _pos_sumsq(px, py, pz, src, dst):
    mesh = plsc.VectorSubcoreMesh(core_axis_name="c", subcore_axis_name="s")
    return pl.kernel(
        _pos_sumsq_body,
        out_type=jax.ShapeDtypeStruct((E_PAD,), jnp.float32),
        mesh=mesh,
        scratch_types=[
            pltpu.VMEM((N_PAD,), jnp.float32),
            pltpu.VMEM((N_PAD,), jnp.float32),
            pltpu.VMEM((B_POS,), jnp.float32),
            pltpu.VMEM((B_POS,), jnp.int32),
            pltpu.VMEM((B_POS,), jnp.int32),
        ],
        compiler_params=pltpu.CompilerParams(needs_layout_passes=False),
    )(px, py, pz, src, dst)


def _node_embed_body(at_ref, rf_ref, pf_ref, aemb_ref, wf_ref, wm_ref,
                     h_ref, g_ref):
    ids = at_ref[:, 0]
    oh = (ids[:, None] == lax.broadcasted_iota(jnp.int32, (BN, 100), 1))
    a_emb = jnp.dot(oh.astype(jnp.float32), aemb_ref[...],
                    preferred_element_type=jnp.float32,
                    precision=lax.Precision.HIGHEST)
    af_r = jnp.dot(rf_ref[...], wf_ref[...], preferred_element_type=jnp.float32,
                   precision=lax.Precision.HIGHEST)
    af_p = jnp.dot(pf_ref[...], wf_ref[...], preferred_element_type=jnp.float32,
                   precision=lax.Precision.HIGHEST)
    z = jnp.concatenate([a_emb + af_r, af_p - af_r], axis=-1)
    h_ref[...] = z
    g = jnp.dot(z, wm_ref[...], preferred_element_type=jnp.float32,
                precision=lax.Precision.HIGHEST)
    g_ref[0] = g[:, :32]
    g_ref[1] = g[:, 32:]


def _node_embed(at, rf, pf, atom_emb, w_feat, wm0):
    grid = N_PAD // BN
    return pl.pallas_call(
        _node_embed_body,
        grid=(grid,),
        in_specs=[
            pl.BlockSpec((BN, 1), lambda i: (i, 0)),
            pl.BlockSpec((BN, FEAT), lambda i: (i, 0)),
            pl.BlockSpec((BN, FEAT), lambda i: (i, 0)),
            pl.BlockSpec((100, 32), lambda i: (0, 0)),
            pl.BlockSpec((FEAT, 32), lambda i: (0, 0)),
            pl.BlockSpec((HID, HID), lambda i: (0, 0)),
        ],
        out_specs=[
            pl.BlockSpec((BN, HID), lambda i: (i, 0)),
            pl.BlockSpec((2, BN, 32), lambda i: (0, i, 0)),
        ],
        out_shape=[
            jax.ShapeDtypeStruct((N_PAD, HID), jnp.float32),
            jax.ShapeDtypeStruct((2, N_PAD, 32), jnp.float32),
        ],
    )(at, rf, pf, atom_emb, w_feat, wm0)


def _edge_base_body(ss_ref, bt_ref, we1_ref, be1_ref, we2_ref, be2_ref,
                    bemb_ref, wc1_ref, bc1_ref, wc2_ref, bc2_ref, wep_ref,
                    ea_ref, ep0_ref):
    pid = pl.program_id(0)
    el = jnp.sqrt(ss_ref[...] + 1e-12)
    hd = jax.nn.relu(el * we1_ref[0][None, :] + be1_ref[0][None, :])
    hd = jnp.dot(hd, we2_ref[...], preferred_element_type=jnp.float32) \
        + be2_ref[0][None, :]
    bt = bt_ref[...]
    bemb = ((bt == 0) * bemb_ref[0][None, :] + (bt == 1) * bemb_ref[1][None, :]
            + (bt == 2) * bemb_ref[2][None, :] + (bt == 3) * bemb_ref[3][None, :])
    attr = hd * bemb
    ea = jax.nn.relu(jnp.dot(attr, wc1_ref[...],
                             preferred_element_type=jnp.float32)
                     + bc1_ref[0][None, :])
    ea = jnp.dot(ea, wc2_ref[...], preferred_element_type=jnp.float32) \
        + bc2_ref[0][None, :]
    eidx = pid * BE + lax.broadcasted_iota(jnp.int32, (BE, 1), 0)
    mask = (eidx < E).astype(jnp.float32)
    ea_ref[...] = ea
    ep = jnp.dot(ea, wep_ref[...], preferred_element_type=jnp.float32) * mask
    ep0_ref[0] = ep[:, 0:32]
    ep0_ref[1] = ep[:, 32:64]


def _edge_base(sumsq, bt, p):
    grid = E_PAD // BE
    wvec = lambda shp: pl.BlockSpec(shp, lambda i: (0, 0))
    wc1s = p["Wc1"][:HID] + p["Wc1"][HID:]
    return pl.pallas_call(
        _edge_base_body,
        grid=(grid,),
        in_specs=[
            pl.BlockSpec((BE, 1), lambda i: (i, 0)),
            pl.BlockSpec((BE, 1), lambda i: (i, 0)),
            wvec((1, HID)), wvec((1, HID)),
            wvec((HID, HID)), wvec((1, HID)),
            wvec((4, HID)),
            wvec((HID, HID)), wvec((1, HID)),
            wvec((HID, HID)), wvec((1, HID)),
            wvec((HID, HID)),
        ],
        out_specs=[
            pl.BlockSpec((BE, HID), lambda i: (i, 0)),
            pl.BlockSpec((2, BE, 32), lambda i: (0, i, 0)),
        ],
        out_shape=[
            jax.ShapeDtypeStruct((E_PAD, HID), jnp.float32),
            jax.ShapeDtypeStruct((2, E_PAD, 32), jnp.float32),
        ],
    )(sumsq, bt, p["We1"], p["be1"].reshape(1, HID), p["We2"],
      p["be2"].reshape(1, HID), p["bond_emb"][:4], wc1s,
      p["bc1"].reshape(1, HID), p["Wc2"], p["bc2"].reshape(1, HID), p["Wep0"])


def _edge_ep12_body(ea_ref, wep_ref, ep1_ref, ep2_ref):
    pid = pl.program_id(0)
    eidx = pid * BE + lax.broadcasted_iota(jnp.int32, (BE, 1), 0)
    mask = (eidx < E).astype(jnp.float32)
    ep = jnp.dot(ea_ref[...], wep_ref[...],
                 preferred_element_type=jnp.float32) * mask
    ep1_ref[0] = ep[:, 0:32]
    ep1_ref[1] = ep[:, 32:64]
    ep2_ref[0] = ep[:, 64:96]
    ep2_ref[1] = ep[:, 96:128]


def _edge_ep12(ea, p):
    grid = E_PAD // BE
    ep_spec = pl.BlockSpec((2, BE, 32), lambda i: (0, i, 0))
    ep_shape = jax.ShapeDtypeStruct((2, E_PAD, 32), jnp.float32)
    wep12 = jnp.concatenate([p["Wep1"], p["Wep2"]], axis=1)
    return pl.pallas_call(
        _edge_ep12_body,
        grid=(grid,),
        in_specs=[
            pl.BlockSpec((BE, HID), lambda i: (i, 0)),
            pl.BlockSpec((HID, 2 * HID), lambda i: (0, 0)),
        ],
        out_specs=[ep_spec, ep_spec],
        out_shape=[ep_shape, ep_shape],
    )(ea, wep12)


def _conv_body(g_ref, ep_ref, src_ref, dst_ref, agg_ref,
               accum, gbuf, ebuf, isbuf, idbuf,
               sem_g0, sem_g1, sem_e0, sem_e1, sem_s0, sem_s1):
    c = lax.axis_index("c")
    s_id = lax.axis_index("s")
    rows_per_tile = N_PAD // NS
    chunk = E_PAD // NS
    sbsz = IB_CONV * B_CONV
    nsb = chunk // sbsz
    sem_g = (sem_g0, sem_g1)
    sem_e = (sem_e0, sem_e1)
    sem_s = (sem_s0, sem_s1)

    def zrow(j, _):
        gbuf[0, j, pl.ds(0, LANES)] = jnp.zeros((LANES,), jnp.float32)
        gbuf[0, j, pl.ds(LANES, LANES)] = jnp.zeros((LANES,), jnp.float32)
        return 0

    lax.fori_loop(0, B_CONV, zrow, 0)
    r0 = s_id * rows_per_tile
    nfull = rows_per_tile // B_CONV
    rem = rows_per_tile - nfull * B_CONV

    def zcp(k, _):
        pltpu.sync_copy(gbuf.at[0], accum.at[pl.ds(r0 + k * B_CONV, B_CONV)])
        return 0

    lax.fori_loop(0, nfull, zcp, 0)
    if rem:
        pltpu.sync_copy(gbuf.at[0, pl.ds(0, rem)],
                        accum.at[pl.ds(r0 + nfull * B_CONV, rem)])
    plsc.subcore_barrier()

    def sblock(sb, _):
        row0 = s_id * (chunk // B_CONV) + sb * IB_CONV
        e_base = s_id * chunk + sb * sbsz
        pltpu.sync_copy(src_ref.at[pl.ds(row0, IB_CONV)], isbuf)
        pltpu.sync_copy(dst_ref.at[pl.ds(row0, IB_CONV)], idbuf)

        def issue(k):
            buf = k % 2
            pltpu.async_copy(g_ref.at[c].at[isbuf.at[k]], gbuf.at[buf],
                             sem_g[buf])
            pltpu.async_copy(
                ep_ref.at[c, pl.ds(e_base + k * B_CONV, B_CONV)],
                ebuf.at[buf], sem_e[buf])

        def wait_in(k):
            buf = k % 2
            pltpu.make_async_copy(g_ref.at[c].at[isbuf.at[k]], gbuf.at[buf],
                                  sem_g[buf]).wait()
            pltpu.make_async_copy(
                ep_ref.at[c, pl.ds(e_base + k * B_CONV, B_CONV)],
                ebuf.at[buf], sem_e[buf]).wait()

        def mul(k):
            buf = k % 2

            def body(j, _):
                lo = pl.ds(0, LANES)
                hi = pl.ds(LANES, LANES)
                gbuf[buf, j, lo] = gbuf[buf, j, lo] * ebuf[buf, j, lo]
                gbuf[buf, j, hi] = gbuf[buf, j, hi] * ebuf[buf, j, hi]
                return 0

            lax.fori_loop(0, B_CONV, body, 0)

        def scatter(k):
            buf = k % 2
            pltpu.async_copy(gbuf.at[buf], accum.at[idbuf.at[k]], sem_s[buf],
                             add=True)

        def wait_scatter(k):
            buf = k % 2
            pltpu.make_async_copy(gbuf.at[buf], accum.at[idbuf.at[k]],
                                  sem_s[buf]).wait()

        issue(0)
        for k in range(IB_CONV):
            wait_in(k)
            if k >= 1:
                wait_scatter(k - 1)
            if k + 1 < IB_CONV:
                issue(k + 1)
            mul(k)
            scatter(k)
        wait_scatter(IB_CONV - 1)
        return 0

    lax.fori_loop(0, nsb, sblock, 0)
    plsc.subcore_barrier()
    pltpu.sync_copy(accum.at[pl.ds(r0, rows_per_tile)],
                    agg_ref.at[c, pl.ds(r0, rows_per_tile)])


def _conv_pass(g, ep, src2, dst2):
    mesh = plsc.VectorSubcoreMesh(core_axis_name="c", subcore_axis_name="s")
    return pl.kernel(
        _conv_body,
        out_type=jax.ShapeDtypeStruct((2, N_PAD, 32), jnp.float32),
        mesh=mesh,
        scratch_types=[
            pltpu.VMEM_SHARED((N_PAD, 32), jnp.float32),
            pltpu.VMEM((2, B_CONV, 32), jnp.float32),
            pltpu.VMEM((2, B_CONV, 32), jnp.float32),
            pltpu.VMEM((IB_CONV, B_CONV), jnp.int32),
            pltpu.VMEM((IB_CONV, B_CONV), jnp.int32),
            pltpu.SemaphoreType.DMA, pltpu.SemaphoreType.DMA,
            pltpu.SemaphoreType.DMA, pltpu.SemaphoreType.DMA,
            pltpu.SemaphoreType.DMA, pltpu.SemaphoreType.DMA,
        ],
        compiler_params=pltpu.CompilerParams(
            needs_layout_passes=False, use_tc_tiling_on_sc=False),
    )(g, ep, src2, dst2)


def _node_update_body(h_ref, agg_ref, wu_ref, bu_ref, wn_ref, hn_ref, g_ref):
    aggc = jnp.concatenate([agg_ref[0], agg_ref[1]], axis=-1)
    hn = h_ref[...] + jax.nn.relu(
        jnp.dot(aggc, wu_ref[...], preferred_element_type=jnp.float32, precision=lax.Precision.HIGHEST)
        + bu_ref[0][None, :])
    hn_ref[...] = hn
    g = jnp.dot(hn, wn_ref[...], preferred_element_type=jnp.float32, precision=lax.Precision.HIGHEST)
    g_ref[0] = g[:, :32]
    g_ref[1] = g[:, 32:]


def _node_update(h, agg, wu, bu, wnext):
    grid = N_PAD // BN
    return pl.pallas_call(
        _node_update_body,
        grid=(grid,),
        in_specs=[
            pl.BlockSpec((BN, HID), lambda i: (i, 0)),
            pl.BlockSpec((2, BN, 32), lambda i: (0, i, 0)),
            pl.BlockSpec((HID, HID), lambda i: (0, 0)),
            pl.BlockSpec((1, HID), lambda i: (0, 0)),
            pl.BlockSpec((HID, HID), lambda i: (0, 0)),
        ],
        out_specs=[
            pl.BlockSpec((BN, HID), lambda i: (i, 0)),
            pl.BlockSpec((2, BN, 32), lambda i: (0, i, 0)),
        ],
        out_shape=[
            jax.ShapeDtypeStruct((N_PAD, HID), jnp.float32),
            jax.ShapeDtypeStruct((2, N_PAD, 32), jnp.float32),
        ],
    )(h, agg, wu, bu.reshape(1, HID), wnext)


def _pair_body(h_ref, src_ref, dst_ref, prod_ref,
               sbuf, dbuf, obuf, isbuf, idbuf,
               sem_a0, sem_a1, sem_b0, sem_b1, sem_w0, sem_w1):
    c = lax.axis_index("c")
    s_id = lax.axis_index("s")
    chunk = E_PAD // NS
    sbsz = IB_PAIR * B_PAIR
    nsb = chunk // sbsz
    sem_a = (sem_a0, sem_a1)
    sem_b = (sem_b0, sem_b1)
    sem_w = (sem_w0, sem_w1)

    def sblock(sb, _):
        row0 = s_id * (chunk // B_PAIR) + sb * IB_PAIR
        e_base = s_id * chunk + sb * sbsz
        pltpu.sync_copy(src_ref.at[pl.ds(row0, IB_PAIR)], isbuf)
        pltpu.sync_copy(dst_ref.at[pl.ds(row0, IB_PAIR)], idbuf)

        def issue(k):
            buf = k % 2
            pltpu.async_copy(h_ref.at[c].at[isbuf.at[k]], sbuf.at[buf],
                             sem_a[buf])
            pltpu.async_copy(h_ref.at[c].at[idbuf.at[k]], dbuf.at[buf],
                             sem_b[buf])

        def wait_in(k):
            buf = k % 2
            pltpu.make_async_copy(h_ref.at[c].at[isbuf.at[k]], sbuf.at[buf],
                                  sem_a[buf]).wait()
            pltpu.make_async_copy(h_ref.at[c].at[idbuf.at[k]], dbuf.at[buf],
                                  sem_b[buf]).wait()

        def mul(k):
            buf = k % 2

            def body(j, _):
                lo = pl.ds(0, LANES)
                hi = pl.ds(LANES, LANES)
                obuf[buf, j, lo] = sbuf[buf, j, lo] * dbuf[buf, j, lo]
                obuf[buf, j, hi] = sbuf[buf, j, hi] * dbuf[buf, j, hi]
                return 0

            lax.fori_loop(0, B_PAIR, body, 0)

        def wr(k):
            buf = k % 2
            pltpu.async_copy(
                obuf.at[buf],
                prod_ref.at[c, pl.ds(e_base + k * B_PAIR, B_PAIR)],
                sem_w[buf])

        def wait_wr(k):
            buf = k % 2
            pltpu.make_async_copy(
                obuf.at[buf],
                prod_ref.at[c, pl.ds(e_base + k * B_PAIR, B_PAIR)],
                sem_w[buf]).wait()

        issue(0)
        for k in range(IB_PAIR):
            wait_in(k)
            if k + 1 < IB_PAIR:
                issue(k + 1)
            if k >= 2:
                wait_wr(k - 2)
            mul(k)
            wr(k)
        wait_wr(IB_PAIR - 2)
        wait_wr(IB_PAIR - 1)
        return 0

    lax.fori_loop(0, nsb, sblock, 0)


def _pair_pass(h_split, src2, dst2):
    mesh = plsc.VectorSubcoreMesh(core_axis_name="c", subcore_axis_name="s")
    return pl.kernel(
        _pair_body,
        out_type=jax.ShapeDtypeStruct((2, E_PAD, 32), jnp.float32),
        mesh=mesh,
        scratch_types=[
            pltpu.VMEM((2, B_PAIR, 32), jnp.float32),
            pltpu.VMEM((2, B_PAIR, 32), jnp.float32),
            pltpu.VMEM((2, B_PAIR, 32), jnp.float32),
            pltpu.VMEM((IB_PAIR, B_PAIR), jnp.int32),
            pltpu.VMEM((IB_PAIR, B_PAIR), jnp.int32),
            pltpu.SemaphoreType.DMA, pltpu.SemaphoreType.DMA,
            pltpu.SemaphoreType.DMA, pltpu.SemaphoreType.DMA,
            pltpu.SemaphoreType.DMA, pltpu.SemaphoreType.DMA,
        ],
        compiler_params=pltpu.CompilerParams(
            needs_layout_passes=False, use_tc_tiling_on_sc=False),
    )(h_split, src2, dst2)


BE_H = 800


def _head_body(prod_ref, ea_ref, ss_ref, wo1_ref, bo1_ref, wo2_ref, bo2_ref,
               wo3_ref, bo3_ref, out_ref, el_ref):
    el_ref[...] = jnp.sqrt(ss_ref[...] + 1e-12)
    hh = jnp.concatenate([prod_ref[0], prod_ref[1], ea_ref[...]], axis=-1)
    o = jax.nn.relu(jnp.dot(hh, wo1_ref[...],
                            preferred_element_type=jnp.float32)
                    + bo1_ref[0][None, :])
    o = jax.nn.relu(jnp.dot(o, wo2_ref[...],
                            preferred_element_type=jnp.float32)
                    + bo2_ref[0][None, :])
    out_ref[...] = jnp.dot(o, wo3_ref[...],
                           preferred_element_type=jnp.float32) \
        + bo3_ref[0][None, :]


def _head(prod, ea, sumsq, p):
    grid = E // BE_H
    wvec = lambda shp: pl.BlockSpec(shp, lambda i: (0, 0))
    return pl.pallas_call(
        _head_body,
        grid=(grid,),
        in_specs=[
            pl.BlockSpec((2, BE_H, 32), lambda i: (0, i, 0)),
            pl.BlockSpec((BE_H, HID), lambda i: (i, 0)),
            pl.BlockSpec((BE_H, 1), lambda i: (i, 0)),
            wvec((2 * HID, HID)), wvec((1, HID)),
            wvec((HID, 32)), wvec((1, 32)),
            wvec((32, 3)), wvec((1, 3)),
        ],
        out_specs=[
            pl.BlockSpec((BE_H, 3), lambda i: (i, 0)),
            pl.BlockSpec((BE_H, 1), lambda i: (i, 0)),
        ],
        out_shape=[
            jax.ShapeDtypeStruct((E, 3), jnp.float32),
            jax.ShapeDtypeStruct((E, 1), jnp.float32),
        ],
    )(prod, ea, sumsq, p["Wo1"], p["bo1"].reshape(1, HID), p["Wo2"],
      p["bo2"].reshape(1, 32), p["Wo3"], p["bo3"].reshape(1, 3))


def kernel(atom_type, r_feat, p_feat, rtsp, pos_N_3, bond_index, bond_type,
           batch, time_step, params):
    p = params
    at = jnp.pad(atom_type.astype(jnp.int32), (0, N_PAD - N)).reshape(N_PAD, 1)
    rf = jnp.pad(r_feat, ((0, N_PAD - N), (0, 0)))
    pf = jnp.pad(p_feat, ((0, N_PAD - N), (0, 0)))
    pos_t = jnp.pad(pos_N_3, ((0, N_PAD - N), (0, 0))).T
    px, py, pz = pos_t[0], pos_t[1], pos_t[2]
    src = jnp.pad(bond_index[0].astype(jnp.int32), (0, E_PAD - E))
    dst = jnp.pad(bond_index[1].astype(jnp.int32), (0, E_PAD - E))
    bt = jnp.pad(bond_type.astype(jnp.int32), (0, E_PAD - E)).reshape(E_PAD, 1)

    src_c = src.reshape(E_PAD // B_CONV, B_CONV)
    dst_c = dst.reshape(E_PAD // B_CONV, B_CONV)
    src_p = src.reshape(E_PAD // B_PAIR, B_PAIR)
    dst_p = dst.reshape(E_PAD // B_PAIR, B_PAIR)

    h, g = _node_embed(at, rf, pf, p["atom_emb"], p["W_feat"], p["Wm0"])
    sumsq = _pos_sumsq(px, py, pz, src, dst).reshape(E_PAD, 1)
    ea, ep0 = _edge_base(sumsq, bt, p)
    ep1, ep2 = _edge_ep12(ea, p)

    eye = jnp.eye(HID, dtype=jnp.float32)
    for i, ep in enumerate((ep0, ep1, ep2)):
        agg = _conv_pass(g, ep, src_c, dst_c)
        wnext = p["Wm%d" % (i + 1)] if i < 2 else eye
        h, g = _node_update(h, agg, p["Wu%d" % i], p["bu%d" % i], wnext)

    prod = _pair_pass(g, src_p, dst_p)
    edge_inv, el = _head(prod, ea, sumsq, p)

    return edge_inv, bond_index, el


# speedup vs baseline: 1.0193x; 1.0020x over previous
import functools

import jax
import jax.numpy as jnp
from jax import lax
from jax.experimental import pallas as pl
from jax.experimental.pallas import tpu as pltpu, tpu_sc as plsc

N = 50000
E = 800000
HID = 64
FEAT = 28

N_PAD = 50176
E_PAD = 819200

NC = 2
NS = 16
LANES = 16

B_POS = 3200
B_CONV = 128
IB_CONV = 25
B_PAIR = 512
IB_PAIR = 10

BE = 1024
BN = 512


def _pos_sumsq_body(px_ref, py_ref, pz_ref, src_ref, dst_ref, out_ref,
                    tab_a, tab_b, sbuf, ibuf_s, ibuf_d):
    wid = lax.axis_index("s") * NC + lax.axis_index("c")
    chunk = E_PAD // (NC * NS)
    nblk = chunk // B_POS
    base = wid * chunk

    pltpu.sync_copy(px_ref, tab_a)
    pltpu.sync_copy(py_ref, tab_b)

    def blk1(b, _):
        e0 = base + b * B_POS
        pltpu.sync_copy(src_ref.at[pl.ds(e0, B_POS)], ibuf_s)
        pltpu.sync_copy(dst_ref.at[pl.ds(e0, B_POS)], ibuf_d)

        def inner(j, _):
            sl = pl.ds(j * LANES, LANES)
            isv = ibuf_s[sl]
            idv = ibuf_d[sl]
            dx = plsc.load_gather(tab_a, [idv]) - plsc.load_gather(tab_a, [isv])
            dy = plsc.load_gather(tab_b, [idv]) - plsc.load_gather(tab_b, [isv])
            sbuf[sl] = dx * dx + dy * dy
            return 0

        lax.fori_loop(0, B_POS // LANES, inner, 0)
        pltpu.sync_copy(sbuf, out_ref.at[pl.ds(e0, B_POS)])
        return 0

    lax.fori_loop(0, nblk, blk1, 0)

    pltpu.sync_copy(pz_ref, tab_a)

    def blk2(b, _):
        e0 = base + b * B_POS
        pltpu.sync_copy(src_ref.at[pl.ds(e0, B_POS)], ibuf_s)
        pltpu.sync_copy(dst_ref.at[pl.ds(e0, B_POS)], ibuf_d)
        pltpu.sync_copy(out_ref.at[pl.ds(e0, B_POS)], sbuf)

        def inner(j, _):
            sl = pl.ds(j * LANES, LANES)
            dz = (plsc.load_gather(tab_a, [ibuf_d[sl]])
                  - plsc.load_gather(tab_a, [ibuf_s[sl]]))
            sbuf[sl] = sbuf[sl] + dz * dz
            return 0

        lax.fori_loop(0, B_POS // LANES, inner, 0)
        pltpu.sync_copy(sbuf, out_ref.at[pl.ds(e0, B_POS)])
        return 0

    lax.fori_loop(0, nblk, blk2, 0)


def _pos_sumsq(px, py, pz, src, dst):
    mesh = plsc.VectorSubcoreMesh(core_axis_name="c", subcore_axis_name="s")
    return pl.kernel(
        _pos_sumsq_body,
        out_type=jax.ShapeDtypeStruct((E_PAD,), jnp.float32),
        mesh=mesh,
        scratch_types=[
            pltpu.VMEM((N_PAD,), jnp.float32),
            pltpu.VMEM((N_PAD,), jnp.float32),
            pltpu.VMEM((B_POS,), jnp.float32),
            pltpu.VMEM((B_POS,), jnp.int32),
            pltpu.VMEM((B_POS,), jnp.int32),
        ],
        compiler_params=pltpu.CompilerParams(needs_layout_passes=False),
    )(px, py, pz, src, dst)


def _node_embed_body(at_ref, rf_ref, pf_ref, aemb_ref, wf_ref, wm_ref,
                     h_ref, g_ref):
    ids = at_ref[:, 0]
    oh = (ids[:, None] == lax.broadcasted_iota(jnp.int32, (BN, 100), 1))
    a_emb = jnp.dot(oh.astype(jnp.float32), aemb_ref[...],
                    preferred_element_type=jnp.float32,
                    precision=lax.Precision.HIGHEST)
    af_r = jnp.dot(rf_ref[...], wf_ref[...], preferred_element_type=jnp.float32,
                   precision=lax.Precision.HIGHEST)
    af_p = jnp.dot(pf_ref[...], wf_ref[...], preferred_element_type=jnp.float32,
                   precision=lax.Precision.HIGHEST)
    z = jnp.concatenate([a_emb + af_r, af_p - af_r], axis=-1)
    h_ref[...] = z
    g = jnp.dot(z, wm_ref[...], preferred_element_type=jnp.float32,
                precision=lax.Precision.HIGHEST)
    g_ref[0] = g[:, :32]
    g_ref[1] = g[:, 32:]


def _node_embed(at, rf, pf, atom_emb, w_feat, wm0):
    grid = N_PAD // BN
    return pl.pallas_call(
        _node_embed_body,
        grid=(grid,),
        in_specs=[
            pl.BlockSpec((BN, 1), lambda i: (i, 0)),
            pl.BlockSpec((BN, FEAT), lambda i: (i, 0)),
            pl.BlockSpec((BN, FEAT), lambda i: (i, 0)),
            pl.BlockSpec((100, 32), lambda i: (0, 0)),
            pl.BlockSpec((FEAT, 32), lambda i: (0, 0)),
            pl.BlockSpec((HID, HID), lambda i: (0, 0)),
        ],
        out_specs=[
            pl.BlockSpec((BN, HID), lambda i: (i, 0)),
            pl.BlockSpec((2, BN, 32), lambda i: (0, i, 0)),
        ],
        out_shape=[
            jax.ShapeDtypeStruct((N_PAD, HID), jnp.float32),
            jax.ShapeDtypeStruct((2, N_PAD, 32), jnp.float32),
        ],
    )(at, rf, pf, atom_emb, w_feat, wm0)


def _edge_base_body(ss_ref, bt_ref, we1_ref, be1_ref, we2_ref, be2_ref,
                    bemb_ref, wc1_ref, bc1_ref, wc2_ref, bc2_ref, wep_ref,
                    ea_ref, ep0_ref):
    pid = pl.program_id(0)
    el = jnp.sqrt(ss_ref[...] + 1e-12)
    hd = jax.nn.relu(el * we1_ref[0][None, :] + be1_ref[0][None, :])
    hd = jnp.dot(hd, we2_ref[...], preferred_element_type=jnp.float32) \
        + be2_ref[0][None, :]
    bt = bt_ref[...]
    bemb = ((bt == 0) * bemb_ref[0][None, :] + (bt == 1) * bemb_ref[1][None, :]
            + (bt == 2) * bemb_ref[2][None, :] + (bt == 3) * bemb_ref[3][None, :])
    attr = hd * bemb
    ea = jax.nn.relu(jnp.dot(attr, wc1_ref[...],
                             preferred_element_type=jnp.float32)
                     + bc1_ref[0][None, :])
    ea = jnp.dot(ea, wc2_ref[...], preferred_element_type=jnp.float32) \
        + bc2_ref[0][None, :]
    eidx = pid * BE + lax.broadcasted_iota(jnp.int32, (BE, 1), 0)
    mask = (eidx < E).astype(jnp.float32)
    ea_ref[...] = ea
    ep = jnp.dot(ea, wep_ref[...], preferred_element_type=jnp.float32) * mask
    ep0_ref[0] = ep[:, 0:32]
    ep0_ref[1] = ep[:, 32:64]


def _edge_base(sumsq, bt, p):
    grid = E_PAD // BE
    wvec = lambda shp: pl.BlockSpec(shp, lambda i: (0, 0))
    wc1s = p["Wc1"][:HID] + p["Wc1"][HID:]
    return pl.pallas_call(
        _edge_base_body,
        grid=(grid,),
        in_specs=[
            pl.BlockSpec((BE, 1), lambda i: (i, 0)),
            pl.BlockSpec((BE, 1), lambda i: (i, 0)),
            wvec((1, HID)), wvec((1, HID)),
            wvec((HID, HID)), wvec((1, HID)),
            wvec((4, HID)),
            wvec((HID, HID)), wvec((1, HID)),
            wvec((HID, HID)), wvec((1, HID)),
            wvec((HID, HID)),
        ],
        out_specs=[
            pl.BlockSpec((BE, HID), lambda i: (i, 0)),
            pl.BlockSpec((2, BE, 32), lambda i: (0, i, 0)),
        ],
        out_shape=[
            jax.ShapeDtypeStruct((E_PAD, HID), jnp.float32),
            jax.ShapeDtypeStruct((2, E_PAD, 32), jnp.float32),
        ],
    )(sumsq, bt, p["We1"], p["be1"].reshape(1, HID), p["We2"],
      p["be2"].reshape(1, HID), p["bond_emb"][:4], wc1s,
      p["bc1"].reshape(1, HID), p["Wc2"], p["bc2"].reshape(1, HID), p["Wep0"])


def _edge_ep12_body(ea_ref, wep_ref, ep1_ref, ep2_ref):
    pid = pl.program_id(0)
    eidx = pid * BE + lax.broadcasted_iota(jnp.int32, (BE, 1), 0)
    mask = (eidx < E).astype(jnp.float32)
    ep = jnp.dot(ea_ref[...], wep_ref[...],
                 preferred_element_type=jnp.float32) * mask
    ep1_ref[0] = ep[:, 0:32]
    ep1_ref[1] = ep[:, 32:64]
    ep2_ref[0] = ep[:, 64:96]
    ep2_ref[1] = ep[:, 96:128]


def _edge_ep12(ea, p):
    grid = E_PAD // BE
    ep_spec = pl.BlockSpec((2, BE, 32), lambda i: (0, i, 0))
    ep_shape = jax.ShapeDtypeStruct((2, E_PAD, 32), jnp.float32)
    wep12 = jnp.concatenate([p["Wep1"], p["Wep2"]], axis=1)
    return pl.pallas_call(
        _edge_ep12_body,
        grid=(grid,),
        in_specs=[
            pl.BlockSpec((BE, HID), lambda i: (i, 0)),
            pl.BlockSpec((HID, 2 * HID), lambda i: (0, 0)),
        ],
        out_specs=[ep_spec, ep_spec],
        out_shape=[ep_shape, ep_shape],
    )(ea, wep12)


def _conv_body(g_ref, ep_ref, src_ref, dst_ref, agg_ref,
               accum, gbuf, ebuf, isbuf, idbuf,
               sem_g0, sem_g1, sem_e0, sem_e1, sem_s0, sem_s1):
    c = lax.axis_index("c")
    s_id = lax.axis_index("s")
    rows_per_tile = N_PAD // NS
    chunk = E_PAD // NS
    sbsz = IB_CONV * B_CONV
    nsb = chunk // sbsz
    sem_g = (sem_g0, sem_g1)
    sem_e = (sem_e0, sem_e1)
    sem_s = (sem_s0, sem_s1)

    def zrow(j, _):
        gbuf[0, j, pl.ds(0, LANES)] = jnp.zeros((LANES,), jnp.float32)
        gbuf[0, j, pl.ds(LANES, LANES)] = jnp.zeros((LANES,), jnp.float32)
        return 0

    lax.fori_loop(0, B_CONV, zrow, 0)
    r0 = s_id * rows_per_tile
    nfull = rows_per_tile // B_CONV
    rem = rows_per_tile - nfull * B_CONV

    def zcp(k, _):
        pltpu.sync_copy(gbuf.at[0], accum.at[pl.ds(r0 + k * B_CONV, B_CONV)])
        return 0

    lax.fori_loop(0, nfull, zcp, 0)
    if rem:
        pltpu.sync_copy(gbuf.at[0, pl.ds(0, rem)],
                        accum.at[pl.ds(r0 + nfull * B_CONV, rem)])
    plsc.subcore_barrier()

    def sblock(sb, _):
        row0 = s_id * (chunk // B_CONV) + sb * IB_CONV
        e_base = s_id * chunk + sb * sbsz
        pltpu.sync_copy(src_ref.at[pl.ds(row0, IB_CONV)], isbuf)
        pltpu.sync_copy(dst_ref.at[pl.ds(row0, IB_CONV)], idbuf)

        def issue(k):
            buf = k % 2
            pltpu.async_copy(g_ref.at[c].at[isbuf.at[k]], gbuf.at[buf],
                             sem_g[buf])
            pltpu.async_copy(
                ep_ref.at[c, pl.ds(e_base + k * B_CONV, B_CONV)],
                ebuf.at[buf], sem_e[buf])

        def wait_in(k):
            buf = k % 2
            pltpu.make_async_copy(g_ref.at[c].at[isbuf.at[k]], gbuf.at[buf],
                                  sem_g[buf]).wait()
            pltpu.make_async_copy(
                ep_ref.at[c, pl.ds(e_base + k * B_CONV, B_CONV)],
                ebuf.at[buf], sem_e[buf]).wait()

        def mul(k):
            buf = k % 2

            def body(j, _):
                lo = pl.ds(0, LANES)
                hi = pl.ds(LANES, LANES)
                gbuf[buf, j, lo] = gbuf[buf, j, lo] * ebuf[buf, j, lo]
                gbuf[buf, j, hi] = gbuf[buf, j, hi] * ebuf[buf, j, hi]
                return 0

            lax.fori_loop(0, B_CONV, body, 0)

        def scatter(k):
            buf = k % 2
            pltpu.async_copy(gbuf.at[buf], accum.at[idbuf.at[k]], sem_s[buf],
                             add=True)

        def wait_scatter(k):
            buf = k % 2
            pltpu.make_async_copy(gbuf.at[buf], accum.at[idbuf.at[k]],
                                  sem_s[buf]).wait()

        issue(0)
        for k in range(IB_CONV):
            wait_in(k)
            if k >= 1:
                wait_scatter(k - 1)
            if k + 1 < IB_CONV:
                issue(k + 1)
            mul(k)
            scatter(k)
        wait_scatter(IB_CONV - 1)
        return 0

    lax.fori_loop(0, nsb, sblock, 0)
    plsc.subcore_barrier()
    pltpu.sync_copy(accum.at[pl.ds(r0, rows_per_tile)],
                    agg_ref.at[c, pl.ds(r0, rows_per_tile)])


def _conv_pass(g, ep, src2, dst2):
    mesh = plsc.VectorSubcoreMesh(core_axis_name="c", subcore_axis_name="s")
    return pl.kernel(
        _conv_body,
        out_type=jax.ShapeDtypeStruct((2, N_PAD, 32), jnp.float32),
        mesh=mesh,
        scratch_types=[
            pltpu.VMEM_SHARED((N_PAD, 32), jnp.float32),
            pltpu.VMEM((2, B_CONV, 32), jnp.float32),
            pltpu.VMEM((2, B_CONV, 32), jnp.float32),
            pltpu.VMEM((IB_CONV, B_CONV), jnp.int32),
            pltpu.VMEM((IB_CONV, B_CONV), jnp.int32),
            pltpu.SemaphoreType.DMA, pltpu.SemaphoreType.DMA,
            pltpu.SemaphoreType.DMA, pltpu.SemaphoreType.DMA,
            pltpu.SemaphoreType.DMA, pltpu.SemaphoreType.DMA,
        ],
        compiler_params=pltpu.CompilerParams(
            needs_layout_passes=False, use_tc_tiling_on_sc=False),
    )(g, ep, src2, dst2)


def _node_update_body(h_ref, agg_ref, wu_ref, bu_ref, wn_ref, hn_ref, g_ref):
    aggc = jnp.concatenate([agg_ref[0], agg_ref[1]], axis=-1)
    hn = h_ref[...] + jax.nn.relu(
        jnp.dot(aggc, wu_ref[...], preferred_element_type=jnp.float32, precision=lax.Precision.HIGHEST)
        + bu_ref[0][None, :])
    hn_ref[...] = hn
    g = jnp.dot(hn, wn_ref[...], preferred_element_type=jnp.float32, precision=lax.Precision.HIGHEST)
    g_ref[0] = g[:, :32]
    g_ref[1] = g[:, 32:]


def _node_update(h, agg, wu, bu, wnext):
    grid = N_PAD // BN
    return pl.pallas_call(
        _node_update_body,
        grid=(grid,),
        in_specs=[
            pl.BlockSpec((BN, HID), lambda i: (i, 0)),
            pl.BlockSpec((2, BN, 32), lambda i: (0, i, 0)),
            pl.BlockSpec((HID, HID), lambda i: (0, 0)),
            pl.BlockSpec((1, HID), lambda i: (0, 0)),
            pl.BlockSpec((HID, HID), lambda i: (0, 0)),
        ],
        out_specs=[
            pl.BlockSpec((BN, HID), lambda i: (i, 0)),
            pl.BlockSpec((2, BN, 32), lambda i: (0, i, 0)),
        ],
        out_shape=[
            jax.ShapeDtypeStruct((N_PAD, HID), jnp.float32),
            jax.ShapeDtypeStruct((2, N_PAD, 32), jnp.float32),
        ],
    )(h, agg, wu, bu.reshape(1, HID), wnext)


def _pair_body(h_ref, src_ref, dst_ref, prod_ref,
               sbuf, dbuf, obuf, isbuf, idbuf,
               sem_a0, sem_a1, sem_b0, sem_b1, sem_w0, sem_w1):
    c = lax.axis_index("c")
    s_id = lax.axis_index("s")
    chunk = E_PAD // NS
    sbsz = IB_PAIR * B_PAIR
    nsb = chunk // sbsz
    sem_a = (sem_a0, sem_a1)
    sem_b = (sem_b0, sem_b1)
    sem_w = (sem_w0, sem_w1)

    def sblock(sb, _):
        row0 = s_id * (chunk // B_PAIR) + sb * IB_PAIR
        e_base = s_id * chunk + sb * sbsz
        pltpu.sync_copy(src_ref.at[pl.ds(row0, IB_PAIR)], isbuf)
        pltpu.sync_copy(dst_ref.at[pl.ds(row0, IB_PAIR)], idbuf)

        def issue(k):
            buf = k % 2
            pltpu.async_copy(h_ref.at[c].at[isbuf.at[k]], sbuf.at[buf],
                             sem_a[buf])
            pltpu.async_copy(h_ref.at[c].at[idbuf.at[k]], dbuf.at[buf],
                             sem_b[buf])

        def wait_in(k):
            buf = k % 2
            pltpu.make_async_copy(h_ref.at[c].at[isbuf.at[k]], sbuf.at[buf],
                                  sem_a[buf]).wait()
            pltpu.make_async_copy(h_ref.at[c].at[idbuf.at[k]], dbuf.at[buf],
                                  sem_b[buf]).wait()

        def mul(k):
            buf = k % 2

            def body(j, _):
                lo = pl.ds(0, LANES)
                hi = pl.ds(LANES, LANES)
                obuf[buf, j, lo] = sbuf[buf, j, lo] * dbuf[buf, j, lo]
                obuf[buf, j, hi] = sbuf[buf, j, hi] * dbuf[buf, j, hi]
                return 0

            lax.fori_loop(0, B_PAIR, body, 0)

        def wr(k):
            buf = k % 2
            pltpu.async_copy(
                obuf.at[buf],
                prod_ref.at[c, pl.ds(e_base + k * B_PAIR, B_PAIR)],
                sem_w[buf])

        def wait_wr(k):
            buf = k % 2
            pltpu.make_async_copy(
                obuf.at[buf],
                prod_ref.at[c, pl.ds(e_base + k * B_PAIR, B_PAIR)],
                sem_w[buf]).wait()

        issue(0)
        for k in range(IB_PAIR):
            wait_in(k)
            if k + 1 < IB_PAIR:
                issue(k + 1)
            if k >= 2:
                wait_wr(k - 2)
            mul(k)
            wr(k)
        wait_wr(IB_PAIR - 2)
        wait_wr(IB_PAIR - 1)
        return 0

    lax.fori_loop(0, nsb, sblock, 0)


def _pair_pass(h_split, src2, dst2):
    mesh = plsc.VectorSubcoreMesh(core_axis_name="c", subcore_axis_name="s")
    return pl.kernel(
        _pair_body,
        out_type=jax.ShapeDtypeStruct((2, E_PAD, 32), jnp.float32),
        mesh=mesh,
        scratch_types=[
            pltpu.VMEM((2, B_PAIR, 32), jnp.float32),
            pltpu.VMEM((2, B_PAIR, 32), jnp.float32),
            pltpu.VMEM((2, B_PAIR, 32), jnp.float32),
            pltpu.VMEM((IB_PAIR, B_PAIR), jnp.int32),
            pltpu.VMEM((IB_PAIR, B_PAIR), jnp.int32),
            pltpu.SemaphoreType.DMA, pltpu.SemaphoreType.DMA,
            pltpu.SemaphoreType.DMA, pltpu.SemaphoreType.DMA,
            pltpu.SemaphoreType.DMA, pltpu.SemaphoreType.DMA,
        ],
        compiler_params=pltpu.CompilerParams(
            needs_layout_passes=False, use_tc_tiling_on_sc=False),
    )(h_split, src2, dst2)


BE_H = 800


def _head_body(prod_ref, ea_ref, ss_ref, wo1_ref, bo1_ref, wo2_ref, bo2_ref,
               wo3_ref, bo3_ref, out_ref, el_ref):
    el_ref[...] = jnp.sqrt(ss_ref[...] + 1e-12)
    hh = jnp.concatenate([prod_ref[0], prod_ref[1], ea_ref[...]], axis=-1)
    o = jax.nn.relu(jnp.dot(hh, wo1_ref[...],
                            preferred_element_type=jnp.float32)
                    + bo1_ref[0][None, :])
    o = jax.nn.relu(jnp.dot(o, wo2_ref[...],
                            preferred_element_type=jnp.float32)
                    + bo2_ref[0][None, :])
    out_ref[...] = jnp.dot(o, wo3_ref[...],
                           preferred_element_type=jnp.float32) \
        + bo3_ref[0][None, :]


def _head(prod, ea, sumsq, p):
    grid = E // BE_H
    wvec = lambda shp: pl.BlockSpec(shp, lambda i: (0, 0))
    return pl.pallas_call(
        _head_body,
        grid=(grid,),
        in_specs=[
            pl.BlockSpec((2, BE_H, 32), lambda i: (0, i, 0)),
            pl.BlockSpec((BE_H, HID), lambda i: (i, 0)),
            pl.BlockSpec((BE_H, 1), lambda i: (i, 0)),
            wvec((2 * HID, HID)), wvec((1, HID)),
            wvec((HID, 32)), wvec((1, 32)),
            wvec((32, 3)), wvec((1, 3)),
        ],
        out_specs=[
            pl.BlockSpec((BE_H, 3), lambda i: (i, 0)),
            pl.BlockSpec((BE_H, 1), lambda i: (i, 0)),
        ],
        out_shape=[
            jax.ShapeDtypeStruct((E, 3), jnp.float32),
            jax.ShapeDtypeStruct((E, 1), jnp.float32),
        ],
    )(prod, ea, sumsq, p["Wo1"], p["bo1"].reshape(1, HID), p["Wo2"],
      p["bo2"].reshape(1, 32), p["Wo3"], p["bo3"].reshape(1, 3))


def kernel(atom_type, r_feat, p_feat, rtsp, pos_N_3, bond_index, bond_type,
           batch, time_step, params):
    p = params
    at = jnp.pad(atom_type.astype(jnp.int32), (0, N_PAD - N)).reshape(N_PAD, 1)
    rf = jnp.pad(r_feat, ((0, N_PAD - N), (0, 0)))
    pf = jnp.pad(p_feat, ((0, N_PAD - N), (0, 0)))
    pos_t = jnp.pad(pos_N_3, ((0, N_PAD - N), (0, 0))).T
    px, py, pz = pos_t[0], pos_t[1], pos_t[2]
    src = jnp.pad(bond_index[0].astype(jnp.int32), (0, E_PAD - E))
    dst = jnp.pad(bond_index[1].astype(jnp.int32), (0, E_PAD - E))
    bt = jnp.pad(bond_type.astype(jnp.int32), (0, E_PAD - E)).reshape(E_PAD, 1)

    src_c = src.reshape(E_PAD // B_CONV, B_CONV)
    dst_c = dst.reshape(E_PAD // B_CONV, B_CONV)
    src_p = src.reshape(E_PAD // B_PAIR, B_PAIR)
    dst_p = dst.reshape(E_PAD // B_PAIR, B_PAIR)

    h, g = _node_embed(at, rf, pf, p["atom_emb"], p["W_feat"], p["Wm0"])
    sumsq = _pos_sumsq(px, py, pz, src, dst).reshape(E_PAD, 1)
    ea, ep0 = _edge_base(sumsq, bt, p)
    ep1, ep2 = _edge_ep12(ea, p)

    eye = jnp.eye(HID, dtype=jnp.float32)
    for i, ep in enumerate((ep0, ep1, ep2)):
        agg = _conv_pass(g, ep, src_c, dst_c)
        wnext = p["Wm%d" % (i + 1)] if i < 2 else eye
        h, g = _node_update(h, agg, p["Wu%d" % i], p["bu%d" % i], wnext)

    prod = _pair_pass(g, src_p, dst_p)
    edge_inv, el = _head(prod, ea, sumsq, p)

    return edge_inv, bond_index, el


# speedup vs baseline: 1.0278x; 1.0084x over previous
import functools

import jax
import jax.numpy as jnp
from jax import lax
from jax.experimental import pallas as pl
from jax.experimental.pallas import tpu as pltpu, tpu_sc as plsc

N = 50000
E = 800000
HID = 64
FEAT = 28

N_PAD = 50176
E_PAD = 819200

NC = 2
NS = 16
LANES = 16

B_POS = 3200
B_CONV = 128
IB_CONV = 16
B_PAIR = 512
IB_PAIR = 10

BE = 1024
BN = 512


def _pos_sumsq_body(px_ref, py_ref, pz_ref, src_ref, dst_ref, out_ref,
                    tab_a, tab_b, sbuf, ibuf_s, ibuf_d):
    wid = lax.axis_index("s") * NC + lax.axis_index("c")
    chunk = E_PAD // (NC * NS)
    nblk = chunk // B_POS
    base = wid * chunk

    pltpu.sync_copy(px_ref, tab_a)
    pltpu.sync_copy(py_ref, tab_b)

    def blk1(b, _):
        e0 = base + b * B_POS
        pltpu.sync_copy(src_ref.at[pl.ds(e0, B_POS)], ibuf_s)
        pltpu.sync_copy(dst_ref.at[pl.ds(e0, B_POS)], ibuf_d)

        def inner(j, _):
            sl = pl.ds(j * LANES, LANES)
            isv = ibuf_s[sl]
            idv = ibuf_d[sl]
            dx = plsc.load_gather(tab_a, [idv]) - plsc.load_gather(tab_a, [isv])
            dy = plsc.load_gather(tab_b, [idv]) - plsc.load_gather(tab_b, [isv])
            sbuf[sl] = dx * dx + dy * dy
            return 0

        lax.fori_loop(0, B_POS // LANES, inner, 0)
        pltpu.sync_copy(sbuf, out_ref.at[pl.ds(e0, B_POS)])
        return 0

    lax.fori_loop(0, nblk, blk1, 0)

    pltpu.sync_copy(pz_ref, tab_a)

    def blk2(b, _):
        e0 = base + b * B_POS
        pltpu.sync_copy(src_ref.at[pl.ds(e0, B_POS)], ibuf_s)
        pltpu.sync_copy(dst_ref.at[pl.ds(e0, B_POS)], ibuf_d)
        pltpu.sync_copy(out_ref.at[pl.ds(e0, B_POS)], sbuf)

        def inner(j, _):
            sl = pl.ds(j * LANES, LANES)
            dz = (plsc.load_gather(tab_a, [ibuf_d[sl]])
                  - plsc.load_gather(tab_a, [ibuf_s[sl]]))
            sbuf[sl] = sbuf[sl] + dz * dz
            return 0

        lax.fori_loop(0, B_POS // LANES, inner, 0)
        pltpu.sync_copy(sbuf, out_ref.at[pl.ds(e0, B_POS)])
        return 0

    lax.fori_loop(0, nblk, blk2, 0)


def _pos_sumsq(px, py, pz, src, dst):
    mesh = plsc.VectorSubcoreMesh(core_axis_name="c", subcore_axis_name="s")
    return pl.kernel(
        _pos_sumsq_body,
        out_type=jax.ShapeDtypeStruct((E_PAD,), jnp.float32),
        mesh=mesh,
        scratch_types=[
            pltpu.VMEM((N_PAD,), jnp.float32),
            pltpu.VMEM((N_PAD,), jnp.float32),
            pltpu.VMEM((B_POS,), jnp.float32),
            pltpu.VMEM((B_POS,), jnp.int32),
            pltpu.VMEM((B_POS,), jnp.int32),
        ],
        compiler_params=pltpu.CompilerParams(needs_layout_passes=False),
    )(px, py, pz, src, dst)


def _node_embed_body(at_ref, rf_ref, pf_ref, aemb_ref, wf_ref, wm_ref,
                     h_ref, g_ref):
    ids = at_ref[:, 0]
    oh = (ids[:, None] == lax.broadcasted_iota(jnp.int32, (BN, 100), 1))
    a_emb = jnp.dot(oh.astype(jnp.float32), aemb_ref[...],
                    preferred_element_type=jnp.float32,
                    precision=lax.Precision.HIGHEST)
    af_r = jnp.dot(rf_ref[...], wf_ref[...], preferred_element_type=jnp.float32,
                   precision=lax.Precision.HIGHEST)
    af_p = jnp.dot(pf_ref[...], wf_ref[...], preferred_element_type=jnp.float32,
                   precision=lax.Precision.HIGHEST)
    z = jnp.concatenate([a_emb + af_r, af_p - af_r], axis=-1)
    h_ref[...] = z
    g = jnp.dot(z, wm_ref[...], preferred_element_type=jnp.float32,
                precision=lax.Precision.HIGHEST)
    g_ref[0] = g[:, :32]
    g_ref[1] = g[:, 32:]


def _node_embed(at, rf, pf, atom_emb, w_feat, wm0):
    grid = N_PAD // BN
    return pl.pallas_call(
        _node_embed_body,
        grid=(grid,),
        in_specs=[
            pl.BlockSpec((BN, 1), lambda i: (i, 0)),
            pl.BlockSpec((BN, FEAT), lambda i: (i, 0)),
            pl.BlockSpec((BN, FEAT), lambda i: (i, 0)),
            pl.BlockSpec((100, 32), lambda i: (0, 0)),
            pl.BlockSpec((FEAT, 32), lambda i: (0, 0)),
            pl.BlockSpec((HID, HID), lambda i: (0, 0)),
        ],
        out_specs=[
            pl.BlockSpec((BN, HID), lambda i: (i, 0)),
            pl.BlockSpec((2, BN, 32), lambda i: (0, i, 0)),
        ],
        out_shape=[
            jax.ShapeDtypeStruct((N_PAD, HID), jnp.float32),
            jax.ShapeDtypeStruct((2, N_PAD, 32), jnp.float32),
        ],
    )(at, rf, pf, atom_emb, w_feat, wm0)


def _edge_base_body(ss_ref, bt_ref, we1_ref, be1_ref, we2_ref, be2_ref,
                    bemb_ref, wc1_ref, bc1_ref, wc2_ref, bc2_ref, wep_ref,
                    ea_ref, ep0_ref):
    pid = pl.program_id(0)
    el = jnp.sqrt(ss_ref[...] + 1e-12)
    hd = jax.nn.relu(el * we1_ref[0][None, :] + be1_ref[0][None, :])
    hd = jnp.dot(hd, we2_ref[...], preferred_element_type=jnp.float32) \
        + be2_ref[0][None, :]
    bt = bt_ref[...]
    bemb = ((bt == 0) * bemb_ref[0][None, :] + (bt == 1) * bemb_ref[1][None, :]
            + (bt == 2) * bemb_ref[2][None, :] + (bt == 3) * bemb_ref[3][None, :])
    attr = hd * bemb
    ea = jax.nn.relu(jnp.dot(attr, wc1_ref[...],
                             preferred_element_type=jnp.float32)
                     + bc1_ref[0][None, :])
    ea = jnp.dot(ea, wc2_ref[...], preferred_element_type=jnp.float32) \
        + bc2_ref[0][None, :]
    eidx = pid * BE + lax.broadcasted_iota(jnp.int32, (BE, 1), 0)
    mask = (eidx < E).astype(jnp.float32)
    ea_ref[...] = ea
    ep = jnp.dot(ea, wep_ref[...], preferred_element_type=jnp.float32) * mask
    ep0_ref[0] = ep[:, 0:32]
    ep0_ref[1] = ep[:, 32:64]


def _edge_base(sumsq, bt, p):
    grid = E_PAD // BE
    wvec = lambda shp: pl.BlockSpec(shp, lambda i: (0, 0))
    wc1s = p["Wc1"][:HID] + p["Wc1"][HID:]
    return pl.pallas_call(
        _edge_base_body,
        grid=(grid,),
        in_specs=[
            pl.BlockSpec((BE, 1), lambda i: (i, 0)),
            pl.BlockSpec((BE, 1), lambda i: (i, 0)),
            wvec((1, HID)), wvec((1, HID)),
            wvec((HID, HID)), wvec((1, HID)),
            wvec((4, HID)),
            wvec((HID, HID)), wvec((1, HID)),
            wvec((HID, HID)), wvec((1, HID)),
            wvec((HID, HID)),
        ],
        out_specs=[
            pl.BlockSpec((BE, HID), lambda i: (i, 0)),
            pl.BlockSpec((2, BE, 32), lambda i: (0, i, 0)),
        ],
        out_shape=[
            jax.ShapeDtypeStruct((E_PAD, HID), jnp.float32),
            jax.ShapeDtypeStruct((2, E_PAD, 32), jnp.float32),
        ],
    )(sumsq, bt, p["We1"], p["be1"].reshape(1, HID), p["We2"],
      p["be2"].reshape(1, HID), p["bond_emb"][:4], wc1s,
      p["bc1"].reshape(1, HID), p["Wc2"], p["bc2"].reshape(1, HID), p["Wep0"])


def _edge_ep12_body(ea_ref, wep_ref, ep1_ref, ep2_ref):
    pid = pl.program_id(0)
    eidx = pid * BE + lax.broadcasted_iota(jnp.int32, (BE, 1), 0)
    mask = (eidx < E).astype(jnp.float32)
    ep = jnp.dot(ea_ref[...], wep_ref[...],
                 preferred_element_type=jnp.float32) * mask
    ep1_ref[0] = ep[:, 0:32]
    ep1_ref[1] = ep[:, 32:64]
    ep2_ref[0] = ep[:, 64:96]
    ep2_ref[1] = ep[:, 96:128]


def _edge_ep12(ea, p):
    grid = E_PAD // BE
    ep_spec = pl.BlockSpec((2, BE, 32), lambda i: (0, i, 0))
    ep_shape = jax.ShapeDtypeStruct((2, E_PAD, 32), jnp.float32)
    wep12 = jnp.concatenate([p["Wep1"], p["Wep2"]], axis=1)
    return pl.pallas_call(
        _edge_ep12_body,
        grid=(grid,),
        in_specs=[
            pl.BlockSpec((BE, HID), lambda i: (i, 0)),
            pl.BlockSpec((HID, 2 * HID), lambda i: (0, 0)),
        ],
        out_specs=[ep_spec, ep_spec],
        out_shape=[ep_shape, ep_shape],
    )(ea, wep12)


def _conv_body(g_ref, ep_ref, src_ref, dst_ref, agg_ref,
               accum, gbuf, ebuf, isbuf, idbuf,
               sem_g0, sem_g1, sem_g2, sem_e0, sem_e1,
               sem_s0, sem_s1, sem_s2):
    c = lax.axis_index("c")
    s_id = lax.axis_index("s")
    rows_per_tile = N_PAD // NS
    chunk = E_PAD // NS
    sbsz = IB_CONV * B_CONV
    nsb = chunk // sbsz
    sem_g = (sem_g0, sem_g1, sem_g2)
    sem_e = (sem_e0, sem_e1)
    sem_s = (sem_s0, sem_s1, sem_s2)

    def zrow(j, _):
        gbuf[0, j, pl.ds(0, LANES)] = jnp.zeros((LANES,), jnp.float32)
        gbuf[0, j, pl.ds(LANES, LANES)] = jnp.zeros((LANES,), jnp.float32)
        return 0

    lax.fori_loop(0, B_CONV, zrow, 0)
    r0 = s_id * rows_per_tile
    nfull = rows_per_tile // B_CONV
    rem = rows_per_tile - nfull * B_CONV

    def zcp(k, _):
        pltpu.sync_copy(gbuf.at[0], accum.at[pl.ds(r0 + k * B_CONV, B_CONV)])
        return 0

    lax.fori_loop(0, nfull, zcp, 0)
    if rem:
        pltpu.sync_copy(gbuf.at[0, pl.ds(0, rem)],
                        accum.at[pl.ds(r0 + nfull * B_CONV, rem)])
    plsc.subcore_barrier()

    def sblock(sb, _):
        row0 = s_id * (chunk // B_CONV) + sb * IB_CONV
        e_base = s_id * chunk + sb * sbsz
        pltpu.sync_copy(src_ref.at[pl.ds(row0, IB_CONV)], isbuf)
        pltpu.sync_copy(dst_ref.at[pl.ds(row0, IB_CONV)], idbuf)

        def issue_g(k):
            buf = k % 3
            pltpu.async_copy(g_ref.at[c].at[isbuf.at[k]], gbuf.at[buf],
                             sem_g[buf])

        def issue_e(k):
            buf = k % 2
            pltpu.async_copy(
                ep_ref.at[c, pl.ds(e_base + k * B_CONV, B_CONV)],
                ebuf.at[buf], sem_e[buf])

        def wait_in(k):
            pltpu.make_async_copy(g_ref.at[c].at[isbuf.at[k]],
                                  gbuf.at[k % 3], sem_g[k % 3]).wait()
            pltpu.make_async_copy(
                ep_ref.at[c, pl.ds(e_base + k * B_CONV, B_CONV)],
                ebuf.at[k % 2], sem_e[k % 2]).wait()

        def mul(k):
            gb = k % 3
            eb = k % 2

            def body(j, _):
                lo = pl.ds(0, LANES)
                hi = pl.ds(LANES, LANES)
                gbuf[gb, j, lo] = gbuf[gb, j, lo] * ebuf[eb, j, lo]
                gbuf[gb, j, hi] = gbuf[gb, j, hi] * ebuf[eb, j, hi]
                return 0

            lax.fori_loop(0, B_CONV, body, 0)

        def scatter(k):
            buf = k % 3
            pltpu.async_copy(gbuf.at[buf], accum.at[idbuf.at[k]], sem_s[buf],
                             add=True)

        def wait_scatter(k):
            buf = k % 3
            pltpu.make_async_copy(gbuf.at[buf], accum.at[idbuf.at[k]],
                                  sem_s[buf]).wait()

        issue_g(0)
        issue_g(1)
        issue_e(0)
        for k in range(IB_CONV):
            wait_in(k)
            if k >= 1:
                wait_scatter(k - 1)
            if k + 2 < IB_CONV:
                issue_g(k + 2)
            if k + 1 < IB_CONV:
                issue_e(k + 1)
            mul(k)
            scatter(k)
        wait_scatter(IB_CONV - 1)
        return 0

    lax.fori_loop(0, nsb, sblock, 0)
    plsc.subcore_barrier()
    pltpu.sync_copy(accum.at[pl.ds(r0, rows_per_tile)],
                    agg_ref.at[c, pl.ds(r0, rows_per_tile)])


def _conv_pass(g, ep, src2, dst2):
    mesh = plsc.VectorSubcoreMesh(core_axis_name="c", subcore_axis_name="s")
    return pl.kernel(
        _conv_body,
        out_type=jax.ShapeDtypeStruct((2, N_PAD, 32), jnp.float32),
        mesh=mesh,
        scratch_types=[
            pltpu.VMEM_SHARED((N_PAD, 32), jnp.float32),
            pltpu.VMEM((3, B_CONV, 32), jnp.float32),
            pltpu.VMEM((2, B_CONV, 32), jnp.float32),
            pltpu.VMEM((IB_CONV, B_CONV), jnp.int32),
            pltpu.VMEM((IB_CONV, B_CONV), jnp.int32),
            pltpu.SemaphoreType.DMA, pltpu.SemaphoreType.DMA,
            pltpu.SemaphoreType.DMA, pltpu.SemaphoreType.DMA,
            pltpu.SemaphoreType.DMA, pltpu.SemaphoreType.DMA,
            pltpu.SemaphoreType.DMA, pltpu.SemaphoreType.DMA,
        ],
        compiler_params=pltpu.CompilerParams(
            needs_layout_passes=False, use_tc_tiling_on_sc=False),
    )(g, ep, src2, dst2)


def _node_update_body(h_ref, agg_ref, wu_ref, bu_ref, wn_ref, hn_ref, g_ref):
    aggc = jnp.concatenate([agg_ref[0], agg_ref[1]], axis=-1)
    hn = h_ref[...] + jax.nn.relu(
        jnp.dot(aggc, wu_ref[...], preferred_element_type=jnp.float32, precision=lax.Precision.HIGHEST)
        + bu_ref[0][None, :])
    hn_ref[...] = hn
    g = jnp.dot(hn, wn_ref[...], preferred_element_type=jnp.float32, precision=lax.Precision.HIGHEST)
    g_ref[0] = g[:, :32]
    g_ref[1] = g[:, 32:]


def _node_update(h, agg, wu, bu, wnext):
    grid = N_PAD // BN
    return pl.pallas_call(
        _node_update_body,
        grid=(grid,),
        in_specs=[
            pl.BlockSpec((BN, HID), lambda i: (i, 0)),
            pl.BlockSpec((2, BN, 32), lambda i: (0, i, 0)),
            pl.BlockSpec((HID, HID), lambda i: (0, 0)),
            pl.BlockSpec((1, HID), lambda i: (0, 0)),
            pl.BlockSpec((HID, HID), lambda i: (0, 0)),
        ],
        out_specs=[
            pl.BlockSpec((BN, HID), lambda i: (i, 0)),
            pl.BlockSpec((2, BN, 32), lambda i: (0, i, 0)),
        ],
        out_shape=[
            jax.ShapeDtypeStruct((N_PAD, HID), jnp.float32),
            jax.ShapeDtypeStruct((2, N_PAD, 32), jnp.float32),
        ],
    )(h, agg, wu, bu.reshape(1, HID), wnext)


def _pair_body(h_ref, src_ref, dst_ref, prod_ref,
               sbuf, dbuf, obuf, isbuf, idbuf,
               sem_a0, sem_a1, sem_b0, sem_b1, sem_w0, sem_w1):
    c = lax.axis_index("c")
    s_id = lax.axis_index("s")
    chunk = E_PAD // NS
    sbsz = IB_PAIR * B_PAIR
    nsb = chunk // sbsz
    sem_a = (sem_a0, sem_a1)
    sem_b = (sem_b0, sem_b1)
    sem_w = (sem_w0, sem_w1)

    def sblock(sb, _):
        row0 = s_id * (chunk // B_PAIR) + sb * IB_PAIR
        e_base = s_id * chunk + sb * sbsz
        pltpu.sync_copy(src_ref.at[pl.ds(row0, IB_PAIR)], isbuf)
        pltpu.sync_copy(dst_ref.at[pl.ds(row0, IB_PAIR)], idbuf)

        def issue(k):
            buf = k % 2
            pltpu.async_copy(h_ref.at[c].at[isbuf.at[k]], sbuf.at[buf],
                             sem_a[buf])
            pltpu.async_copy(h_ref.at[c].at[idbuf.at[k]], dbuf.at[buf],
                             sem_b[buf])

        def wait_in(k):
            buf = k % 2
            pltpu.make_async_copy(h_ref.at[c].at[isbuf.at[k]], sbuf.at[buf],
                                  sem_a[buf]).wait()
            pltpu.make_async_copy(h_ref.at[c].at[idbuf.at[k]], dbuf.at[buf],
                                  sem_b[buf]).wait()

        def mul(k):
            buf = k % 2

            def body(j, _):
                lo = pl.ds(0, LANES)
                hi = pl.ds(LANES, LANES)
                obuf[buf, j, lo] = sbuf[buf, j, lo] * dbuf[buf, j, lo]
                obuf[buf, j, hi] = sbuf[buf, j, hi] * dbuf[buf, j, hi]
                return 0

            lax.fori_loop(0, B_PAIR, body, 0)

        def wr(k):
            buf = k % 2
            pltpu.async_copy(
                obuf.at[buf],
                prod_ref.at[c, pl.ds(e_base + k * B_PAIR, B_PAIR)],
                sem_w[buf])

        def wait_wr(k):
            buf = k % 2
            pltpu.make_async_copy(
                obuf.at[buf],
                prod_ref.at[c, pl.ds(e_base + k * B_PAIR, B_PAIR)],
                sem_w[buf]).wait()

        issue(0)
        for k in range(IB_PAIR):
            wait_in(k)
            if k + 1 < IB_PAIR:
                issue(k + 1)
            if k >= 2:
                wait_wr(k - 2)
            mul(k)
            wr(k)
        wait_wr(IB_PAIR - 2)
        wait_wr(IB_PAIR - 1)
        return 0

    lax.fori_loop(0, nsb, sblock, 0)


def _pair_pass(h_split, src2, dst2):
    mesh = plsc.VectorSubcoreMesh(core_axis_name="c", subcore_axis_name="s")
    return pl.kernel(
        _pair_body,
        out_type=jax.ShapeDtypeStruct((2, E_PAD, 32), jnp.float32),
        mesh=mesh,
        scratch_types=[
            pltpu.VMEM((2, B_PAIR, 32), jnp.float32),
            pltpu.VMEM((2, B_PAIR, 32), jnp.float32),
            pltpu.VMEM((2, B_PAIR, 32), jnp.float32),
            pltpu.VMEM((IB_PAIR, B_PAIR), jnp.int32),
            pltpu.VMEM((IB_PAIR, B_PAIR), jnp.int32),
            pltpu.SemaphoreType.DMA, pltpu.SemaphoreType.DMA,
            pltpu.SemaphoreType.DMA, pltpu.SemaphoreType.DMA,
            pltpu.SemaphoreType.DMA, pltpu.SemaphoreType.DMA,
        ],
        compiler_params=pltpu.CompilerParams(
            needs_layout_passes=False, use_tc_tiling_on_sc=False),
    )(h_split, src2, dst2)


BE_H = 800


def _head_body(prod_ref, ea_ref, ss_ref, wo1_ref, bo1_ref, wo2_ref, bo2_ref,
               wo3_ref, bo3_ref, out_ref, el_ref):
    el_ref[...] = jnp.sqrt(ss_ref[...] + 1e-12)
    hh = jnp.concatenate([prod_ref[0], prod_ref[1], ea_ref[...]], axis=-1)
    o = jax.nn.relu(jnp.dot(hh, wo1_ref[...],
                            preferred_element_type=jnp.float32)
                    + bo1_ref[0][None, :])
    o = jax.nn.relu(jnp.dot(o, wo2_ref[...],
                            preferred_element_type=jnp.float32)
                    + bo2_ref[0][None, :])
    out_ref[...] = jnp.dot(o, wo3_ref[...],
                           preferred_element_type=jnp.float32) \
        + bo3_ref[0][None, :]


def _head(prod, ea, sumsq, p):
    grid = E // BE_H
    wvec = lambda shp: pl.BlockSpec(shp, lambda i: (0, 0))
    return pl.pallas_call(
        _head_body,
        grid=(grid,),
        in_specs=[
            pl.BlockSpec((2, BE_H, 32), lambda i: (0, i, 0)),
            pl.BlockSpec((BE_H, HID), lambda i: (i, 0)),
            pl.BlockSpec((BE_H, 1), lambda i: (i, 0)),
            wvec((2 * HID, HID)), wvec((1, HID)),
            wvec((HID, 32)), wvec((1, 32)),
            wvec((32, 3)), wvec((1, 3)),
        ],
        out_specs=[
            pl.BlockSpec((BE_H, 3), lambda i: (i, 0)),
            pl.BlockSpec((BE_H, 1), lambda i: (i, 0)),
        ],
        out_shape=[
            jax.ShapeDtypeStruct((E, 3), jnp.float32),
            jax.ShapeDtypeStruct((E, 1), jnp.float32),
        ],
    )(prod, ea, sumsq, p["Wo1"], p["bo1"].reshape(1, HID), p["Wo2"],
      p["bo2"].reshape(1, 32), p["Wo3"], p["bo3"].reshape(1, 3))


def kernel(atom_type, r_feat, p_feat, rtsp, pos_N_3, bond_index, bond_type,
           batch, time_step, params):
    p = params
    at = jnp.pad(atom_type.astype(jnp.int32), (0, N_PAD - N)).reshape(N_PAD, 1)
    rf = jnp.pad(r_feat, ((0, N_PAD - N), (0, 0)))
    pf = jnp.pad(p_feat, ((0, N_PAD - N), (0, 0)))
    pos_t = jnp.pad(pos_N_3, ((0, N_PAD - N), (0, 0))).T
    px, py, pz = pos_t[0], pos_t[1], pos_t[2]
    src = jnp.pad(bond_index[0].astype(jnp.int32), (0, E_PAD - E))
    dst = jnp.pad(bond_index[1].astype(jnp.int32), (0, E_PAD - E))
    bt = jnp.pad(bond_type.astype(jnp.int32), (0, E_PAD - E)).reshape(E_PAD, 1)

    src_c = src.reshape(E_PAD // B_CONV, B_CONV)
    dst_c = dst.reshape(E_PAD // B_CONV, B_CONV)
    src_p = src.reshape(E_PAD // B_PAIR, B_PAIR)
    dst_p = dst.reshape(E_PAD // B_PAIR, B_PAIR)

    h, g = _node_embed(at, rf, pf, p["atom_emb"], p["W_feat"], p["Wm0"])
    sumsq = _pos_sumsq(px, py, pz, src, dst).reshape(E_PAD, 1)
    ea, ep0 = _edge_base(sumsq, bt, p)
    ep1, ep2 = _edge_ep12(ea, p)

    eye = jnp.eye(HID, dtype=jnp.float32)
    for i, ep in enumerate((ep0, ep1, ep2)):
        agg = _conv_pass(g, ep, src_c, dst_c)
        wnext = p["Wm%d" % (i + 1)] if i < 2 else eye
        h, g = _node_update(h, agg, p["Wu%d" % i], p["bu%d" % i], wnext)

    prod = _pair_pass(g, src_p, dst_p)
    edge_inv, el = _head(prod, ea, sumsq, p)

    return edge_inv, bond_index, el


# speedup vs baseline: 1.0313x; 1.0033x over previous
import functools

import jax
import jax.numpy as jnp
from jax import lax
from jax.experimental import pallas as pl
from jax.experimental.pallas import tpu as pltpu, tpu_sc as plsc

N = 50000
E = 800000
HID = 64
FEAT = 28

N_PAD = 50176
E_PAD = 819200

NC = 2
NS = 16
LANES = 16

B_POS = 3200
B_CONV = 128
IB_CONV = 16
B_PAIR = 320
IB_PAIR = 16

BE = 1024
BN = 512


def _pos_sumsq_body(px_ref, py_ref, pz_ref, src_ref, dst_ref, out_ref,
                    tab_a, tab_b, sbuf, ibuf_s, ibuf_d):
    wid = lax.axis_index("s") * NC + lax.axis_index("c")
    chunk = E_PAD // (NC * NS)
    nblk = chunk // B_POS
    base = wid * chunk

    pltpu.sync_copy(px_ref, tab_a)
    pltpu.sync_copy(py_ref, tab_b)

    def blk1(b, _):
        e0 = base + b * B_POS
        pltpu.sync_copy(src_ref.at[pl.ds(e0, B_POS)], ibuf_s)
        pltpu.sync_copy(dst_ref.at[pl.ds(e0, B_POS)], ibuf_d)

        def inner(j, _):
            sl = pl.ds(j * LANES, LANES)
            isv = ibuf_s[sl]
            idv = ibuf_d[sl]
            dx = plsc.load_gather(tab_a, [idv]) - plsc.load_gather(tab_a, [isv])
            dy = plsc.load_gather(tab_b, [idv]) - plsc.load_gather(tab_b, [isv])
            sbuf[sl] = dx * dx + dy * dy
            return 0

        lax.fori_loop(0, B_POS // LANES, inner, 0)
        pltpu.sync_copy(sbuf, out_ref.at[pl.ds(e0, B_POS)])
        return 0

    lax.fori_loop(0, nblk, blk1, 0)

    pltpu.sync_copy(pz_ref, tab_a)

    def blk2(b, _):
        e0 = base + b * B_POS
        pltpu.sync_copy(src_ref.at[pl.ds(e0, B_POS)], ibuf_s)
        pltpu.sync_copy(dst_ref.at[pl.ds(e0, B_POS)], ibuf_d)
        pltpu.sync_copy(out_ref.at[pl.ds(e0, B_POS)], sbuf)

        def inner(j, _):
            sl = pl.ds(j * LANES, LANES)
            dz = (plsc.load_gather(tab_a, [ibuf_d[sl]])
                  - plsc.load_gather(tab_a, [ibuf_s[sl]]))
            sbuf[sl] = sbuf[sl] + dz * dz
            return 0

        lax.fori_loop(0, B_POS // LANES, inner, 0)
        pltpu.sync_copy(sbuf, out_ref.at[pl.ds(e0, B_POS)])
        return 0

    lax.fori_loop(0, nblk, blk2, 0)


def _pos_sumsq(px, py, pz, src, dst):
    mesh = plsc.VectorSubcoreMesh(core_axis_name="c", subcore_axis_name="s")
    return pl.kernel(
        _pos_sumsq_body,
        out_type=jax.ShapeDtypeStruct((E_PAD,), jnp.float32),
        mesh=mesh,
        scratch_types=[
            pltpu.VMEM((N_PAD,), jnp.float32),
            pltpu.VMEM((N_PAD,), jnp.float32),
            pltpu.VMEM((B_POS,), jnp.float32),
            pltpu.VMEM((B_POS,), jnp.int32),
            pltpu.VMEM((B_POS,), jnp.int32),
        ],
        compiler_params=pltpu.CompilerParams(needs_layout_passes=False),
    )(px, py, pz, src, dst)


def _node_embed_body(at_ref, rf_ref, pf_ref, aemb_ref, wf_ref, wm_ref,
                     h_ref, g_ref):
    ids = at_ref[:, 0]
    oh = (ids[:, None] == lax.broadcasted_iota(jnp.int32, (BN, 100), 1))
    a_emb = jnp.dot(oh.astype(jnp.float32), aemb_ref[...],
                    preferred_element_type=jnp.float32,
                    precision=lax.Precision.HIGHEST)
    af_r = jnp.dot(rf_ref[...], wf_ref[...], preferred_element_type=jnp.float32,
                   precision=lax.Precision.HIGHEST)
    af_p = jnp.dot(pf_ref[...], wf_ref[...], preferred_element_type=jnp.float32,
                   precision=lax.Precision.HIGHEST)
    z = jnp.concatenate([a_emb + af_r, af_p - af_r], axis=-1)
    h_ref[...] = z
    g = jnp.dot(z, wm_ref[...], preferred_element_type=jnp.float32,
                precision=lax.Precision.HIGHEST)
    g_ref[0] = g[:, :32]
    g_ref[1] = g[:, 32:]


def _node_embed(at, rf, pf, atom_emb, w_feat, wm0):
    grid = N_PAD // BN
    return pl.pallas_call(
        _node_embed_body,
        grid=(grid,),
        in_specs=[
            pl.BlockSpec((BN, 1), lambda i: (i, 0)),
            pl.BlockSpec((BN, FEAT), lambda i: (i, 0)),
            pl.BlockSpec((BN, FEAT), lambda i: (i, 0)),
            pl.BlockSpec((100, 32), lambda i: (0, 0)),
            pl.BlockSpec((FEAT, 32), lambda i: (0, 0)),
            pl.BlockSpec((HID, HID), lambda i: (0, 0)),
        ],
        out_specs=[
            pl.BlockSpec((BN, HID), lambda i: (i, 0)),
            pl.BlockSpec((2, BN, 32), lambda i: (0, i, 0)),
        ],
        out_shape=[
            jax.ShapeDtypeStruct((N_PAD, HID), jnp.float32),
            jax.ShapeDtypeStruct((2, N_PAD, 32), jnp.float32),
        ],
    )(at, rf, pf, atom_emb, w_feat, wm0)


def _edge_base_body(ss_ref, bt_ref, we1_ref, be1_ref, we2_ref, be2_ref,
                    bemb_ref, wc1_ref, bc1_ref, wc2_ref, bc2_ref, wep_ref,
                    ea_ref, ep0_ref):
    pid = pl.program_id(0)
    el = jnp.sqrt(ss_ref[...] + 1e-12)
    hd = jax.nn.relu(el * we1_ref[0][None, :] + be1_ref[0][None, :])
    hd = jnp.dot(hd, we2_ref[...], preferred_element_type=jnp.float32) \
        + be2_ref[0][None, :]
    bt = bt_ref[...]
    bemb = ((bt == 0) * bemb_ref[0][None, :] + (bt == 1) * bemb_ref[1][None, :]
            + (bt == 2) * bemb_ref[2][None, :] + (bt == 3) * bemb_ref[3][None, :])
    attr = hd * bemb
    ea = jax.nn.relu(jnp.dot(attr, wc1_ref[...],
                             preferred_element_type=jnp.float32)
                     + bc1_ref[0][None, :])
    ea = jnp.dot(ea, wc2_ref[...], preferred_element_type=jnp.float32) \
        + bc2_ref[0][None, :]
    eidx = pid * BE + lax.broadcasted_iota(jnp.int32, (BE, 1), 0)
    mask = (eidx < E).astype(jnp.float32)
    ea_ref[...] = ea
    ep = jnp.dot(ea, wep_ref[...], preferred_element_type=jnp.float32) * mask
    ep0_ref[0] = ep[:, 0:32]
    ep0_ref[1] = ep[:, 32:64]


def _edge_base(sumsq, bt, p):
    grid = E_PAD // BE
    wvec = lambda shp: pl.BlockSpec(shp, lambda i: (0, 0))
    wc1s = p["Wc1"][:HID] + p["Wc1"][HID:]
    return pl.pallas_call(
        _edge_base_body,
        grid=(grid,),
        in_specs=[
            pl.BlockSpec((BE, 1), lambda i: (i, 0)),
            pl.BlockSpec((BE, 1), lambda i: (i, 0)),
            wvec((1, HID)), wvec((1, HID)),
            wvec((HID, HID)), wvec((1, HID)),
            wvec((4, HID)),
            wvec((HID, HID)), wvec((1, HID)),
            wvec((HID, HID)), wvec((1, HID)),
            wvec((HID, HID)),
        ],
        out_specs=[
            pl.BlockSpec((BE, HID), lambda i: (i, 0)),
            pl.BlockSpec((2, BE, 32), lambda i: (0, i, 0)),
        ],
        out_shape=[
            jax.ShapeDtypeStruct((E_PAD, HID), jnp.float32),
            jax.ShapeDtypeStruct((2, E_PAD, 32), jnp.float32),
        ],
    )(sumsq, bt, p["We1"], p["be1"].reshape(1, HID), p["We2"],
      p["be2"].reshape(1, HID), p["bond_emb"][:4], wc1s,
      p["bc1"].reshape(1, HID), p["Wc2"], p["bc2"].reshape(1, HID), p["Wep0"])


def _edge_ep12_body(ea_ref, wep_ref, ep1_ref, ep2_ref):
    pid = pl.program_id(0)
    eidx = pid * BE + lax.broadcasted_iota(jnp.int32, (BE, 1), 0)
    mask = (eidx < E).astype(jnp.float32)
    ep = jnp.dot(ea_ref[...], wep_ref[...],
                 preferred_element_type=jnp.float32) * mask
    ep1_ref[0] = ep[:, 0:32]
    ep1_ref[1] = ep[:, 32:64]
    ep2_ref[0] = ep[:, 64:96]
    ep2_ref[1] = ep[:, 96:128]


def _edge_ep12(ea, p):
    grid = E_PAD // BE
    ep_spec = pl.BlockSpec((2, BE, 32), lambda i: (0, i, 0))
    ep_shape = jax.ShapeDtypeStruct((2, E_PAD, 32), jnp.float32)
    wep12 = jnp.concatenate([p["Wep1"], p["Wep2"]], axis=1)
    return pl.pallas_call(
        _edge_ep12_body,
        grid=(grid,),
        in_specs=[
            pl.BlockSpec((BE, HID), lambda i: (i, 0)),
            pl.BlockSpec((HID, 2 * HID), lambda i: (0, 0)),
        ],
        out_specs=[ep_spec, ep_spec],
        out_shape=[ep_shape, ep_shape],
    )(ea, wep12)


def _conv_body(g_ref, ep_ref, src_ref, dst_ref, agg_ref,
               accum, gbuf, ebuf, isbuf, idbuf,
               sem_g0, sem_g1, sem_g2, sem_e0, sem_e1,
               sem_s0, sem_s1, sem_s2):
    c = lax.axis_index("c")
    s_id = lax.axis_index("s")
    rows_per_tile = N_PAD // NS
    chunk = E_PAD // NS
    sbsz = IB_CONV * B_CONV
    nsb = chunk // sbsz
    sem_g = (sem_g0, sem_g1, sem_g2)
    sem_e = (sem_e0, sem_e1)
    sem_s = (sem_s0, sem_s1, sem_s2)

    def zrow(j, _):
        gbuf[0, j, pl.ds(0, LANES)] = jnp.zeros((LANES,), jnp.float32)
        gbuf[0, j, pl.ds(LANES, LANES)] = jnp.zeros((LANES,), jnp.float32)
        return 0

    lax.fori_loop(0, B_CONV, zrow, 0)
    r0 = s_id * rows_per_tile
    nfull = rows_per_tile // B_CONV
    rem = rows_per_tile - nfull * B_CONV

    def zcp(k, _):
        pltpu.sync_copy(gbuf.at[0], accum.at[pl.ds(r0 + k * B_CONV, B_CONV)])
        return 0

    lax.fori_loop(0, nfull, zcp, 0)
    if rem:
        pltpu.sync_copy(gbuf.at[0, pl.ds(0, rem)],
                        accum.at[pl.ds(r0 + nfull * B_CONV, rem)])
    plsc.subcore_barrier()

    def sblock(sb, _):
        row0 = s_id * (chunk // B_CONV) + sb * IB_CONV
        e_base = s_id * chunk + sb * sbsz
        pltpu.sync_copy(src_ref.at[pl.ds(row0, IB_CONV)], isbuf)
        pltpu.sync_copy(dst_ref.at[pl.ds(row0, IB_CONV)], idbuf)

        def issue_g(k):
            buf = k % 3
            pltpu.async_copy(g_ref.at[c].at[isbuf.at[k]], gbuf.at[buf],
                             sem_g[buf])

        def issue_e(k):
            buf = k % 2
            pltpu.async_copy(
                ep_ref.at[c, pl.ds(e_base + k * B_CONV, B_CONV)],
                ebuf.at[buf], sem_e[buf])

        def wait_in(k):
            pltpu.make_async_copy(g_ref.at[c].at[isbuf.at[k]],
                                  gbuf.at[k % 3], sem_g[k % 3]).wait()
            pltpu.make_async_copy(
                ep_ref.at[c, pl.ds(e_base + k * B_CONV, B_CONV)],
                ebuf.at[k % 2], sem_e[k % 2]).wait()

        def mul(k):
            gb = k % 3
            eb = k % 2

            def body(j, _):
                lo = pl.ds(0, LANES)
                hi = pl.ds(LANES, LANES)
                gbuf[gb, j, lo] = gbuf[gb, j, lo] * ebuf[eb, j, lo]
                gbuf[gb, j, hi] = gbuf[gb, j, hi] * ebuf[eb, j, hi]
                return 0

            lax.fori_loop(0, B_CONV, body, 0)

        def scatter(k):
            buf = k % 3
            pltpu.async_copy(gbuf.at[buf], accum.at[idbuf.at[k]], sem_s[buf],
                             add=True)

        def wait_scatter(k):
            buf = k % 3
            pltpu.make_async_copy(gbuf.at[buf], accum.at[idbuf.at[k]],
                                  sem_s[buf]).wait()

        issue_g(0)
        issue_g(1)
        issue_e(0)
        for k in range(IB_CONV):
            wait_in(k)
            if k >= 1:
                wait_scatter(k - 1)
            if k + 2 < IB_CONV:
                issue_g(k + 2)
            if k + 1 < IB_CONV:
                issue_e(k + 1)
            mul(k)
            scatter(k)
        wait_scatter(IB_CONV - 1)
        return 0

    lax.fori_loop(0, nsb, sblock, 0)
    plsc.subcore_barrier()
    pltpu.sync_copy(accum.at[pl.ds(r0, rows_per_tile)],
                    agg_ref.at[c, pl.ds(r0, rows_per_tile)])


def _conv_pass(g, ep, src2, dst2):
    mesh = plsc.VectorSubcoreMesh(core_axis_name="c", subcore_axis_name="s")
    return pl.kernel(
        _conv_body,
        out_type=jax.ShapeDtypeStruct((2, N_PAD, 32), jnp.float32),
        mesh=mesh,
        scratch_types=[
            pltpu.VMEM_SHARED((N_PAD, 32), jnp.float32),
            pltpu.VMEM((3, B_CONV, 32), jnp.float32),
            pltpu.VMEM((2, B_CONV, 32), jnp.float32),
            pltpu.VMEM((IB_CONV, B_CONV), jnp.int32),
            pltpu.VMEM((IB_CONV, B_CONV), jnp.int32),
            pltpu.SemaphoreType.DMA, pltpu.SemaphoreType.DMA,
            pltpu.SemaphoreType.DMA, pltpu.SemaphoreType.DMA,
            pltpu.SemaphoreType.DMA, pltpu.SemaphoreType.DMA,
            pltpu.SemaphoreType.DMA, pltpu.SemaphoreType.DMA,
        ],
        compiler_params=pltpu.CompilerParams(
            needs_layout_passes=False, use_tc_tiling_on_sc=False),
    )(g, ep, src2, dst2)


def _node_update_body(h_ref, agg_ref, wu_ref, bu_ref, wn_ref, hn_ref, g_ref):
    aggc = jnp.concatenate([agg_ref[0], agg_ref[1]], axis=-1)
    hn = h_ref[...] + jax.nn.relu(
        jnp.dot(aggc, wu_ref[...], preferred_element_type=jnp.float32, precision=lax.Precision.HIGHEST)
        + bu_ref[0][None, :])
    hn_ref[...] = hn
    g = jnp.dot(hn, wn_ref[...], preferred_element_type=jnp.float32, precision=lax.Precision.HIGHEST)
    g_ref[0] = g[:, :32]
    g_ref[1] = g[:, 32:]


def _node_update(h, agg, wu, bu, wnext):
    grid = N_PAD // BN
    return pl.pallas_call(
        _node_update_body,
        grid=(grid,),
        in_specs=[
            pl.BlockSpec((BN, HID), lambda i: (i, 0)),
            pl.BlockSpec((2, BN, 32), lambda i: (0, i, 0)),
            pl.BlockSpec((HID, HID), lambda i: (0, 0)),
            pl.BlockSpec((1, HID), lambda i: (0, 0)),
            pl.BlockSpec((HID, HID), lambda i: (0, 0)),
        ],
        out_specs=[
            pl.BlockSpec((BN, HID), lambda i: (i, 0)),
            pl.BlockSpec((2, BN, 32), lambda i: (0, i, 0)),
        ],
        out_shape=[
            jax.ShapeDtypeStruct((N_PAD, HID), jnp.float32),
            jax.ShapeDtypeStruct((2, N_PAD, 32), jnp.float32),
        ],
    )(h, agg, wu, bu.reshape(1, HID), wnext)


def _pair_body(h_ref, src_ref, dst_ref, prod_ref,
               sbuf, dbuf, obuf, isbuf, idbuf,
               sem_a0, sem_a1, sem_a2, sem_b0, sem_b1, sem_b2,
               sem_w0, sem_w1):
    c = lax.axis_index("c")
    s_id = lax.axis_index("s")
    chunk = E_PAD // NS
    sbsz = IB_PAIR * B_PAIR
    nsb = chunk // sbsz
    sem_a = (sem_a0, sem_a1, sem_a2)
    sem_b = (sem_b0, sem_b1, sem_b2)
    sem_w = (sem_w0, sem_w1)

    def sblock(sb, _):
        row0 = s_id * (chunk // B_PAIR) + sb * IB_PAIR
        e_base = s_id * chunk + sb * sbsz
        pltpu.sync_copy(src_ref.at[pl.ds(row0, IB_PAIR)], isbuf)
        pltpu.sync_copy(dst_ref.at[pl.ds(row0, IB_PAIR)], idbuf)

        def issue(k):
            buf = k % 3
            pltpu.async_copy(h_ref.at[c].at[isbuf.at[k]], sbuf.at[buf],
                             sem_a[buf])
            pltpu.async_copy(h_ref.at[c].at[idbuf.at[k]], dbuf.at[buf],
                             sem_b[buf])

        def wait_in(k):
            buf = k % 3
            pltpu.make_async_copy(h_ref.at[c].at[isbuf.at[k]], sbuf.at[buf],
                                  sem_a[buf]).wait()
            pltpu.make_async_copy(h_ref.at[c].at[idbuf.at[k]], dbuf.at[buf],
                                  sem_b[buf]).wait()

        def mul(k):
            buf = k % 3
            ob = k % 2

            def body(j, _):
                lo = pl.ds(0, LANES)
                hi = pl.ds(LANES, LANES)
                obuf[ob, j, lo] = sbuf[buf, j, lo] * dbuf[buf, j, lo]
                obuf[ob, j, hi] = sbuf[buf, j, hi] * dbuf[buf, j, hi]
                return 0

            lax.fori_loop(0, B_PAIR, body, 0)

        def wr(k):
            buf = k % 2
            pltpu.async_copy(
                obuf.at[buf],
                prod_ref.at[c, pl.ds(e_base + k * B_PAIR, B_PAIR)],
                sem_w[buf])

        def wait_wr(k):
            buf = k % 2
            pltpu.make_async_copy(
                obuf.at[buf],
                prod_ref.at[c, pl.ds(e_base + k * B_PAIR, B_PAIR)],
                sem_w[buf]).wait()

        issue(0)
        issue(1)
        for k in range(IB_PAIR):
            wait_in(k)
            if k + 2 < IB_PAIR:
                issue(k + 2)
            if k >= 2:
                wait_wr(k - 2)
            mul(k)
            wr(k)
        wait_wr(IB_PAIR - 2)
        wait_wr(IB_PAIR - 1)
        return 0

    lax.fori_loop(0, nsb, sblock, 0)


def _pair_pass(h_split, src2, dst2):
    mesh = plsc.VectorSubcoreMesh(core_axis_name="c", subcore_axis_name="s")
    return pl.kernel(
        _pair_body,
        out_type=jax.ShapeDtypeStruct((2, E_PAD, 32), jnp.float32),
        mesh=mesh,
        scratch_types=[
            pltpu.VMEM((3, B_PAIR, 32), jnp.float32),
            pltpu.VMEM((3, B_PAIR, 32), jnp.float32),
            pltpu.VMEM((2, B_PAIR, 32), jnp.float32),
            pltpu.VMEM((IB_PAIR, B_PAIR), jnp.int32),
            pltpu.VMEM((IB_PAIR, B_PAIR), jnp.int32),
            pltpu.SemaphoreType.DMA, pltpu.SemaphoreType.DMA,
            pltpu.SemaphoreType.DMA, pltpu.SemaphoreType.DMA,
            pltpu.SemaphoreType.DMA, pltpu.SemaphoreType.DMA,
            pltpu.SemaphoreType.DMA, pltpu.SemaphoreType.DMA,
        ],
        compiler_params=pltpu.CompilerParams(
            needs_layout_passes=False, use_tc_tiling_on_sc=False),
    )(h_split, src2, dst2)


BE_H = 800


def _head_body(prod_ref, ea_ref, ss_ref, wo1_ref, bo1_ref, wo2_ref, bo2_ref,
               wo3_ref, bo3_ref, out_ref, el_ref):
    el_ref[...] = jnp.sqrt(ss_ref[...] + 1e-12)
    hh = jnp.concatenate([prod_ref[0], prod_ref[1], ea_ref[...]], axis=-1)
    o = jax.nn.relu(jnp.dot(hh, wo1_ref[...],
                            preferred_element_type=jnp.float32)
                    + bo1_ref[0][None, :])
    o = jax.nn.relu(jnp.dot(o, wo2_ref[...],
                            preferred_element_type=jnp.float32)
                    + bo2_ref[0][None, :])
    out_ref[...] = jnp.dot(o, wo3_ref[...],
                           preferred_element_type=jnp.float32) \
        + bo3_ref[0][None, :]


def _head(prod, ea, sumsq, p):
    grid = E // BE_H
    wvec = lambda shp: pl.BlockSpec(shp, lambda i: (0, 0))
    return pl.pallas_call(
        _head_body,
        grid=(grid,),
        in_specs=[
            pl.BlockSpec((2, BE_H, 32), lambda i: (0, i, 0)),
            pl.BlockSpec((BE_H, HID), lambda i: (i, 0)),
            pl.BlockSpec((BE_H, 1), lambda i: (i, 0)),
            wvec((2 * HID, HID)), wvec((1, HID)),
            wvec((HID, 32)), wvec((1, 32)),
            wvec((32, 3)), wvec((1, 3)),
        ],
        out_specs=[
            pl.BlockSpec((BE_H, 3), lambda i: (i, 0)),
            pl.BlockSpec((BE_H, 1), lambda i: (i, 0)),
        ],
        out_shape=[
            jax.ShapeDtypeStruct((E, 3), jnp.float32),
            jax.ShapeDtypeStruct((E, 1), jnp.float32),
        ],
    )(prod, ea, sumsq, p["Wo1"], p["bo1"].reshape(1, HID), p["Wo2"],
      p["bo2"].reshape(1, 32), p["Wo3"], p["bo3"].reshape(1, 3))


def kernel(atom_type, r_feat, p_feat, rtsp, pos_N_3, bond_index, bond_type,
           batch, time_step, params):
    p = params
    at = jnp.pad(atom_type.astype(jnp.int32), (0, N_PAD - N)).reshape(N_PAD, 1)
    rf = jnp.pad(r_feat, ((0, N_PAD - N), (0, 0)))
    pf = jnp.pad(p_feat, ((0, N_PAD - N), (0, 0)))
    pos_t = jnp.pad(pos_N_3, ((0, N_PAD - N), (0, 0))).T
    px, py, pz = pos_t[0], pos_t[1], pos_t[2]
    src = jnp.pad(bond_index[0].astype(jnp.int32), (0, E_PAD - E))
    dst = jnp.pad(bond_index[1].astype(jnp.int32), (0, E_PAD - E))
    bt = jnp.pad(bond_type.astype(jnp.int32), (0, E_PAD - E)).reshape(E_PAD, 1)

    src_c = src.reshape(E_PAD // B_CONV, B_CONV)
    dst_c = dst.reshape(E_PAD // B_CONV, B_CONV)
    src_p = src.reshape(E_PAD // B_PAIR, B_PAIR)
    dst_p = dst.reshape(E_PAD // B_PAIR, B_PAIR)

    h, g = _node_embed(at, rf, pf, p["atom_emb"], p["W_feat"], p["Wm0"])
    sumsq = _pos_sumsq(px, py, pz, src, dst).reshape(E_PAD, 1)
    ea, ep0 = _edge_base(sumsq, bt, p)
    ep1, ep2 = _edge_ep12(ea, p)

    eye = jnp.eye(HID, dtype=jnp.float32)
    for i, ep in enumerate((ep0, ep1, ep2)):
        agg = _conv_pass(g, ep, src_c, dst_c)
        wnext = p["Wm%d" % (i + 1)] if i < 2 else eye
        h, g = _node_update(h, agg, p["Wu%d" % i], p["bu%d" % i], wnext)

    prod = _pair_pass(g, src_p, dst_p)
    edge_inv, el = _head(prod, ea, sumsq, p)

    return edge_inv, bond_index, el


# speedup vs baseline: 1.0315x; 1.0002x over previous
import jax
import jax.numpy as jnp
from jax import lax
from jax.experimental import pallas as pl
from jax.experimental.pallas import tpu as pltpu, tpu_sc as plsc

N = 50000
E = 800000
HID = 64
FEAT = 28

N_PAD = 50176
E_PAD = 819200

NC = 2
NS = 16
LANES = 16

B_POS = 3200
B_CONV = 128
IB_CONV = 16
B_PAIR = 320
IB_PAIR = 16

BE = 1024
BN = 512


def _pos_sumsq_body(px_ref, py_ref, pz_ref, src_ref, dst_ref, out_ref,
                    tab_a, tab_b, sbuf, ibuf_s, ibuf_d):
    wid = lax.axis_index("s") * NC + lax.axis_index("c")
    chunk = E_PAD // (NC * NS)
    nblk = chunk // B_POS
    base = wid * chunk

    pltpu.sync_copy(px_ref, tab_a)
    pltpu.sync_copy(py_ref, tab_b)

    def blk1(b, _):
        e0 = base + b * B_POS
        pltpu.sync_copy(src_ref.at[pl.ds(e0, B_POS)], ibuf_s)
        pltpu.sync_copy(dst_ref.at[pl.ds(e0, B_POS)], ibuf_d)

        def inner(j, _):
            sl = pl.ds(j * LANES, LANES)
            isv = ibuf_s[sl]
            idv = ibuf_d[sl]
            dx = plsc.load_gather(tab_a, [idv]) - plsc.load_gather(tab_a, [isv])
            dy = plsc.load_gather(tab_b, [idv]) - plsc.load_gather(tab_b, [isv])
            sbuf[sl] = dx * dx + dy * dy
            return 0

        lax.fori_loop(0, B_POS // LANES, inner, 0)
        pltpu.sync_copy(sbuf, out_ref.at[pl.ds(e0, B_POS)])
        return 0

    lax.fori_loop(0, nblk, blk1, 0)

    pltpu.sync_copy(pz_ref, tab_a)

    def blk2(b, _):
        e0 = base + b * B_POS
        pltpu.sync_copy(src_ref.at[pl.ds(e0, B_POS)], ibuf_s)
        pltpu.sync_copy(dst_ref.at[pl.ds(e0, B_POS)], ibuf_d)
        pltpu.sync_copy(out_ref.at[pl.ds(e0, B_POS)], sbuf)

        def inner(j, _):
            sl = pl.ds(j * LANES, LANES)
            dz = (plsc.load_gather(tab_a, [ibuf_d[sl]])
                  - plsc.load_gather(tab_a, [ibuf_s[sl]]))
            sbuf[sl] = sbuf[sl] + dz * dz
            return 0

        lax.fori_loop(0, B_POS // LANES, inner, 0)
        pltpu.sync_copy(sbuf, out_ref.at[pl.ds(e0, B_POS)])
        return 0

    lax.fori_loop(0, nblk, blk2, 0)


def _pos_sumsq(px, py, pz, src, dst):
    mesh = plsc.VectorSubcoreMesh(core_axis_name="c", subcore_axis_name="s")
    return pl.kernel(
        _pos_sumsq_body,
        out_type=jax.ShapeDtypeStruct((E_PAD,), jnp.float32),
        mesh=mesh,
        scratch_types=[
            pltpu.VMEM((N_PAD,), jnp.float32),
            pltpu.VMEM((N_PAD,), jnp.float32),
            pltpu.VMEM((B_POS,), jnp.float32),
            pltpu.VMEM((B_POS,), jnp.int32),
            pltpu.VMEM((B_POS,), jnp.int32),
        ],
        compiler_params=pltpu.CompilerParams(needs_layout_passes=False),
    )(px, py, pz, src, dst)


def _node_embed_body(at_ref, rf_ref, pf_ref, aemb_ref, wf_ref, wm_ref,
                     h_ref, g_ref):
    ids = at_ref[:, 0]
    oh = (ids[:, None] == lax.broadcasted_iota(jnp.int32, (BN, 100), 1))
    a_emb = jnp.dot(oh.astype(jnp.float32), aemb_ref[...],
                    preferred_element_type=jnp.float32,
                    precision=lax.Precision.HIGHEST)
    af_r = jnp.dot(rf_ref[...], wf_ref[...], preferred_element_type=jnp.float32,
                   precision=lax.Precision.HIGHEST)
    af_p = jnp.dot(pf_ref[...], wf_ref[...], preferred_element_type=jnp.float32,
                   precision=lax.Precision.HIGHEST)
    z = jnp.concatenate([a_emb + af_r, af_p - af_r], axis=-1)
    h_ref[...] = z
    g = jnp.dot(z, wm_ref[...], preferred_element_type=jnp.float32,
                precision=lax.Precision.HIGHEST)
    g_ref[0] = g[:, :32]
    g_ref[1] = g[:, 32:]


def _node_embed(at, rf, pf, atom_emb, w_feat, wm0):
    grid = N_PAD // BN
    return pl.pallas_call(
        _node_embed_body,
        grid=(grid,),
        in_specs=[
            pl.BlockSpec((BN, 1), lambda i: (i, 0)),
            pl.BlockSpec((BN, FEAT), lambda i: (i, 0)),
            pl.BlockSpec((BN, FEAT), lambda i: (i, 0)),
            pl.BlockSpec((100, 32), lambda i: (0, 0)),
            pl.BlockSpec((FEAT, 32), lambda i: (0, 0)),
            pl.BlockSpec((HID, HID), lambda i: (0, 0)),
        ],
        out_specs=[
            pl.BlockSpec((BN, HID), lambda i: (i, 0)),
            pl.BlockSpec((2, BN, 32), lambda i: (0, i, 0)),
        ],
        out_shape=[
            jax.ShapeDtypeStruct((N_PAD, HID), jnp.float32),
            jax.ShapeDtypeStruct((2, N_PAD, 32), jnp.float32),
        ],
    )(at, rf, pf, atom_emb, w_feat, wm0)


def _edge_base_body(ss_ref, bt_ref, we1_ref, be1_ref, we2_ref, be2_ref,
                    bemb_ref, wc1_ref, bc1_ref, wc2_ref, bc2_ref, wep_ref,
                    ea_ref, ep0_ref):
    pid = pl.program_id(0)
    el = jnp.sqrt(ss_ref[...] + 1e-12)
    hd = jax.nn.relu(el * we1_ref[0][None, :] + be1_ref[0][None, :])
    hd = jnp.dot(hd, we2_ref[...], preferred_element_type=jnp.float32) \
        + be2_ref[0][None, :]
    bt = bt_ref[...]
    bemb = ((bt == 0) * bemb_ref[0][None, :] + (bt == 1) * bemb_ref[1][None, :]
            + (bt == 2) * bemb_ref[2][None, :] + (bt == 3) * bemb_ref[3][None, :])
    attr = hd * bemb
    ea = jax.nn.relu(jnp.dot(attr, wc1_ref[...],
                             preferred_element_type=jnp.float32)
                     + bc1_ref[0][None, :])
    ea = jnp.dot(ea, wc2_ref[...], preferred_element_type=jnp.float32) \
        + bc2_ref[0][None, :]
    eidx = pid * BE + lax.broadcasted_iota(jnp.int32, (BE, 1), 0)
    mask = (eidx < E).astype(jnp.float32)
    ea_ref[...] = ea
    ep = jnp.dot(ea, wep_ref[...], preferred_element_type=jnp.float32) * mask
    ep0_ref[0] = ep[:, 0:32]
    ep0_ref[1] = ep[:, 32:64]


def _edge_base(sumsq, bt, p):
    grid = E_PAD // BE
    wvec = lambda shp: pl.BlockSpec(shp, lambda i: (0, 0))
    wc1s = p["Wc1"][:HID] + p["Wc1"][HID:]
    return pl.pallas_call(
        _edge_base_body,
        grid=(grid,),
        in_specs=[
            pl.BlockSpec((BE, 1), lambda i: (i, 0)),
            pl.BlockSpec((BE, 1), lambda i: (i, 0)),
            wvec((1, HID)), wvec((1, HID)),
            wvec((HID, HID)), wvec((1, HID)),
            wvec((4, HID)),
            wvec((HID, HID)), wvec((1, HID)),
            wvec((HID, HID)), wvec((1, HID)),
            wvec((HID, HID)),
        ],
        out_specs=[
            pl.BlockSpec((BE, HID), lambda i: (i, 0)),
            pl.BlockSpec((2, BE, 32), lambda i: (0, i, 0)),
        ],
        out_shape=[
            jax.ShapeDtypeStruct((E_PAD, HID), jnp.float32),
            jax.ShapeDtypeStruct((2, E_PAD, 32), jnp.float32),
        ],
    )(sumsq, bt, p["We1"], p["be1"].reshape(1, HID), p["We2"],
      p["be2"].reshape(1, HID), p["bond_emb"][:4], wc1s,
      p["bc1"].reshape(1, HID), p["Wc2"], p["bc2"].reshape(1, HID), p["Wep0"])


def _edge_ep12_body(ea_ref, wep_ref, ep1_ref, ep2_ref):
    pid = pl.program_id(0)
    eidx = pid * BE + lax.broadcasted_iota(jnp.int32, (BE, 1), 0)
    mask = (eidx < E).astype(jnp.float32)
    ep = jnp.dot(ea_ref[...], wep_ref[...],
                 preferred_element_type=jnp.float32) * mask
    ep1_ref[0] = ep[:, 0:32]
    ep1_ref[1] = ep[:, 32:64]
    ep2_ref[0] = ep[:, 64:96]
    ep2_ref[1] = ep[:, 96:128]


def _edge_ep12(ea, p):
    grid = E_PAD // BE
    ep_spec = pl.BlockSpec((2, BE, 32), lambda i: (0, i, 0))
    ep_shape = jax.ShapeDtypeStruct((2, E_PAD, 32), jnp.float32)
    wep12 = jnp.concatenate([p["Wep1"], p["Wep2"]], axis=1)
    return pl.pallas_call(
        _edge_ep12_body,
        grid=(grid,),
        in_specs=[
            pl.BlockSpec((BE, HID), lambda i: (i, 0)),
            pl.BlockSpec((HID, 2 * HID), lambda i: (0, 0)),
        ],
        out_specs=[ep_spec, ep_spec],
        out_shape=[ep_shape, ep_shape],
    )(ea, wep12)


def _conv_body(g_ref, ep_ref, src_ref, dst_ref, agg_ref,
               accum, gbuf, ebuf, isbuf, idbuf,
               sem_g0, sem_g1, sem_g2, sem_e0, sem_e1,
               sem_s0, sem_s1, sem_s2):
    c = lax.axis_index("c")
    s_id = lax.axis_index("s")
    rows_per_tile = N_PAD // NS
    chunk = E_PAD // NS
    sbsz = IB_CONV * B_CONV
    nsb = chunk // sbsz
    sem_g = (sem_g0, sem_g1, sem_g2)
    sem_e = (sem_e0, sem_e1)
    sem_s = (sem_s0, sem_s1, sem_s2)

    def zrow(j, _):
        gbuf[0, j, pl.ds(0, LANES)] = jnp.zeros((LANES,), jnp.float32)
        gbuf[0, j, pl.ds(LANES, LANES)] = jnp.zeros((LANES,), jnp.float32)
        return 0

    lax.fori_loop(0, B_CONV, zrow, 0)
    r0 = s_id * rows_per_tile
    nfull = rows_per_tile // B_CONV
    rem = rows_per_tile - nfull * B_CONV

    def zcp(k, _):
        pltpu.sync_copy(gbuf.at[0], accum.at[pl.ds(r0 + k * B_CONV, B_CONV)])
        return 0

    lax.fori_loop(0, nfull, zcp, 0)
    if rem:
        pltpu.sync_copy(gbuf.at[0, pl.ds(0, rem)],
                        accum.at[pl.ds(r0 + nfull * B_CONV, rem)])
    plsc.subcore_barrier()

    def sblock(sb, _):
        row0 = s_id * (chunk // B_CONV) + sb * IB_CONV
        e_base = s_id * chunk + sb * sbsz
        pltpu.sync_copy(src_ref.at[pl.ds(row0, IB_CONV)], isbuf)
        pltpu.sync_copy(dst_ref.at[pl.ds(row0, IB_CONV)], idbuf)

        def issue_g(k):
            buf = k % 3
            pltpu.async_copy(g_ref.at[c].at[isbuf.at[k]], gbuf.at[buf],
                             sem_g[buf])

        def issue_e(k):
            buf = k % 2
            pltpu.async_copy(
                ep_ref.at[c, pl.ds(e_base + k * B_CONV, B_CONV)],
                ebuf.at[buf], sem_e[buf])

        def wait_in(k):
            pltpu.make_async_copy(g_ref.at[c].at[isbuf.at[k]],
                                  gbuf.at[k % 3], sem_g[k % 3]).wait()
            pltpu.make_async_copy(
                ep_ref.at[c, pl.ds(e_base + k * B_CONV, B_CONV)],
                ebuf.at[k % 2], sem_e[k % 2]).wait()

        def mul(k):
            gb = k % 3
            eb = k % 2

            def body(j, _):
                lo = pl.ds(0, LANES)
                hi = pl.ds(LANES, LANES)
                gbuf[gb, j, lo] = gbuf[gb, j, lo] * ebuf[eb, j, lo]
                gbuf[gb, j, hi] = gbuf[gb, j, hi] * ebuf[eb, j, hi]
                return 0

            lax.fori_loop(0, B_CONV, body, 0)

        def scatter(k):
            buf = k % 3
            pltpu.async_copy(gbuf.at[buf], accum.at[idbuf.at[k]], sem_s[buf],
                             add=True)

        def wait_scatter(k):
            buf = k % 3
            pltpu.make_async_copy(gbuf.at[buf], accum.at[idbuf.at[k]],
                                  sem_s[buf]).wait()

        issue_g(0)
        issue_g(1)
        issue_e(0)
        for k in range(IB_CONV):
            wait_in(k)
            if k >= 1:
                wait_scatter(k - 1)
            if k + 2 < IB_CONV:
                issue_g(k + 2)
            if k + 1 < IB_CONV:
                issue_e(k + 1)
            mul(k)
            scatter(k)
        wait_scatter(IB_CONV - 1)
        return 0

    lax.fori_loop(0, nsb, sblock, 0)
    plsc.subcore_barrier()
    pltpu.sync_copy(accum.at[pl.ds(r0, rows_per_tile)],
                    agg_ref.at[c, pl.ds(r0, rows_per_tile)])


def _conv_pass(g, ep, src2, dst2):
    mesh = plsc.VectorSubcoreMesh(core_axis_name="c", subcore_axis_name="s")
    return pl.kernel(
        _conv_body,
        out_type=jax.ShapeDtypeStruct((2, N_PAD, 32), jnp.float32),
        mesh=mesh,
        scratch_types=[
            pltpu.VMEM_SHARED((N_PAD, 32), jnp.float32),
            pltpu.VMEM((3, B_CONV, 32), jnp.float32),
            pltpu.VMEM((2, B_CONV, 32), jnp.float32),
            pltpu.VMEM((IB_CONV, B_CONV), jnp.int32),
            pltpu.VMEM((IB_CONV, B_CONV), jnp.int32),
            pltpu.SemaphoreType.DMA, pltpu.SemaphoreType.DMA,
            pltpu.SemaphoreType.DMA, pltpu.SemaphoreType.DMA,
            pltpu.SemaphoreType.DMA, pltpu.SemaphoreType.DMA,
            pltpu.SemaphoreType.DMA, pltpu.SemaphoreType.DMA,
        ],
        compiler_params=pltpu.CompilerParams(
            needs_layout_passes=False, use_tc_tiling_on_sc=False),
    )(g, ep, src2, dst2)


def _node_update_body(h_ref, agg_ref, wu_ref, bu_ref, wn_ref, hn_ref, g_ref):
    aggc = jnp.concatenate([agg_ref[0], agg_ref[1]], axis=-1)
    hn = h_ref[...] + jax.nn.relu(
        jnp.dot(aggc, wu_ref[...], preferred_element_type=jnp.float32, precision=lax.Precision.HIGHEST)
        + bu_ref[0][None, :])
    hn_ref[...] = hn
    g = jnp.dot(hn, wn_ref[...], preferred_element_type=jnp.float32, precision=lax.Precision.HIGHEST)
    g_ref[0] = g[:, :32]
    g_ref[1] = g[:, 32:]


def _node_update(h, agg, wu, bu, wnext):
    grid = N_PAD // BN
    return pl.pallas_call(
        _node_update_body,
        grid=(grid,),
        in_specs=[
            pl.BlockSpec((BN, HID), lambda i: (i, 0)),
            pl.BlockSpec((2, BN, 32), lambda i: (0, i, 0)),
            pl.BlockSpec((HID, HID), lambda i: (0, 0)),
            pl.BlockSpec((1, HID), lambda i: (0, 0)),
            pl.BlockSpec((HID, HID), lambda i: (0, 0)),
        ],
        out_specs=[
            pl.BlockSpec((BN, HID), lambda i: (i, 0)),
            pl.BlockSpec((2, BN, 32), lambda i: (0, i, 0)),
        ],
        out_shape=[
            jax.ShapeDtypeStruct((N_PAD, HID), jnp.float32),
            jax.ShapeDtypeStruct((2, N_PAD, 32), jnp.float32),
        ],
    )(h, agg, wu, bu.reshape(1, HID), wnext)


def _pair_body(h_ref, src_ref, dst_ref, prod_ref,
               sbuf, dbuf, obuf, isbuf, idbuf,
               sem_a0, sem_a1, sem_a2, sem_b0, sem_b1, sem_b2,
               sem_w0, sem_w1):
    c = lax.axis_index("c")
    s_id = lax.axis_index("s")
    chunk = E_PAD // NS
    sbsz = IB_PAIR * B_PAIR
    nsb = chunk // sbsz
    sem_a = (sem_a0, sem_a1, sem_a2)
    sem_b = (sem_b0, sem_b1, sem_b2)
    sem_w = (sem_w0, sem_w1)

    def sblock(sb, _):
        row0 = s_id * (chunk // B_PAIR) + sb * IB_PAIR
        e_base = s_id * chunk + sb * sbsz
        pltpu.sync_copy(src_ref.at[pl.ds(row0, IB_PAIR)], isbuf)
        pltpu.sync_copy(dst_ref.at[pl.ds(row0, IB_PAIR)], idbuf)

        def issue(k):
            buf = k % 3
            pltpu.async_copy(h_ref.at[c].at[isbuf.at[k]], sbuf.at[buf],
                             sem_a[buf])
            pltpu.async_copy(h_ref.at[c].at[idbuf.at[k]], dbuf.at[buf],
                             sem_b[buf])

        def wait_in(k):
            buf = k % 3
            pltpu.make_async_copy(h_ref.at[c].at[isbuf.at[k]], sbuf.at[buf],
                                  sem_a[buf]).wait()
            pltpu.make_async_copy(h_ref.at[c].at[idbuf.at[k]], dbuf.at[buf],
                                  sem_b[buf]).wait()

        def mul(k):
            buf = k % 3
            ob = k % 2

            def body(j, _):
                lo = pl.ds(0, LANES)
                hi = pl.ds(LANES, LANES)
                obuf[ob, j, lo] = sbuf[buf, j, lo] * dbuf[buf, j, lo]
                obuf[ob, j, hi] = sbuf[buf, j, hi] * dbuf[buf, j, hi]
                return 0

            lax.fori_loop(0, B_PAIR, body, 0)

        def wr(k):
            buf = k % 2
            pltpu.async_copy(
                obuf.at[buf],
                prod_ref.at[c, pl.ds(e_base + k * B_PAIR, B_PAIR)],
                sem_w[buf])

        def wait_wr(k):
            buf = k % 2
            pltpu.make_async_copy(
                obuf.at[buf],
                prod_ref.at[c, pl.ds(e_base + k * B_PAIR, B_PAIR)],
                sem_w[buf]).wait()

        issue(0)
        issue(1)
        for k in range(IB_PAIR):
            wait_in(k)
            if k + 2 < IB_PAIR:
                issue(k + 2)
            if k >= 2:
                wait_wr(k - 2)
            mul(k)
            wr(k)
        wait_wr(IB_PAIR - 2)
        wait_wr(IB_PAIR - 1)
        return 0

    lax.fori_loop(0, nsb, sblock, 0)


def _pair_pass(h_split, src2, dst2):
    mesh = plsc.VectorSubcoreMesh(core_axis_name="c", subcore_axis_name="s")
    return pl.kernel(
        _pair_body,
        out_type=jax.ShapeDtypeStruct((2, E_PAD, 32), jnp.float32),
        mesh=mesh,
        scratch_types=[
            pltpu.VMEM((3, B_PAIR, 32), jnp.float32),
            pltpu.VMEM((3, B_PAIR, 32), jnp.float32),
            pltpu.VMEM((2, B_PAIR, 32), jnp.float32),
            pltpu.VMEM((IB_PAIR, B_PAIR), jnp.int32),
            pltpu.VMEM((IB_PAIR, B_PAIR), jnp.int32),
            pltpu.SemaphoreType.DMA, pltpu.SemaphoreType.DMA,
            pltpu.SemaphoreType.DMA, pltpu.SemaphoreType.DMA,
            pltpu.SemaphoreType.DMA, pltpu.SemaphoreType.DMA,
            pltpu.SemaphoreType.DMA, pltpu.SemaphoreType.DMA,
        ],
        compiler_params=pltpu.CompilerParams(
            needs_layout_passes=False, use_tc_tiling_on_sc=False),
    )(h_split, src2, dst2)


BE_H = 800


def _head_body(prod_ref, ea_ref, ss_ref, wo1_ref, bo1_ref, wo2_ref, bo2_ref,
               wo3_ref, bo3_ref, out_ref, el_ref):
    el_ref[...] = jnp.sqrt(ss_ref[...] + 1e-12)
    hh = jnp.concatenate([prod_ref[0], prod_ref[1], ea_ref[...]], axis=-1)
    o = jax.nn.relu(jnp.dot(hh, wo1_ref[...],
                            preferred_element_type=jnp.float32)
                    + bo1_ref[0][None, :])
    o = jax.nn.relu(jnp.dot(o, wo2_ref[...],
                            preferred_element_type=jnp.float32)
                    + bo2_ref[0][None, :])
    out_ref[...] = jnp.dot(o, wo3_ref[...],
                           preferred_element_type=jnp.float32) \
        + bo3_ref[0][None, :]


def _head(prod, ea, sumsq, p):
    grid = E // BE_H
    wvec = lambda shp: pl.BlockSpec(shp, lambda i: (0, 0))
    return pl.pallas_call(
        _head_body,
        grid=(grid,),
        in_specs=[
            pl.BlockSpec((2, BE_H, 32), lambda i: (0, i, 0)),
            pl.BlockSpec((BE_H, HID), lambda i: (i, 0)),
            pl.BlockSpec((BE_H, 1), lambda i: (i, 0)),
            wvec((2 * HID, HID)), wvec((1, HID)),
            wvec((HID, 32)), wvec((1, 32)),
            wvec((32, 3)), wvec((1, 3)),
        ],
        out_specs=[
            pl.BlockSpec((BE_H, 3), lambda i: (i, 0)),
            pl.BlockSpec((BE_H, 1), lambda i: (i, 0)),
        ],
        out_shape=[
            jax.ShapeDtypeStruct((E, 3), jnp.float32),
            jax.ShapeDtypeStruct((E, 1), jnp.float32),
        ],
    )(prod, ea, sumsq, p["Wo1"], p["bo1"].reshape(1, HID), p["Wo2"],
      p["bo2"].reshape(1, 32), p["Wo3"], p["bo3"].reshape(1, 3))


def kernel(atom_type, r_feat, p_feat, rtsp, pos_N_3, bond_index, bond_type,
           batch, time_step, params):
    p = params
    at = jnp.pad(atom_type.astype(jnp.int32), (0, N_PAD - N)).reshape(N_PAD, 1)
    rf = jnp.pad(r_feat, ((0, N_PAD - N), (0, 0)))
    pf = jnp.pad(p_feat, ((0, N_PAD - N), (0, 0)))
    pos_t = jnp.pad(pos_N_3, ((0, N_PAD - N), (0, 0))).T
    px, py, pz = pos_t[0], pos_t[1], pos_t[2]
    src = jnp.pad(bond_index[0].astype(jnp.int32), (0, E_PAD - E))
    dst = jnp.pad(bond_index[1].astype(jnp.int32), (0, E_PAD - E))
    bt = jnp.pad(bond_type.astype(jnp.int32), (0, E_PAD - E)).reshape(E_PAD, 1)

    src_c = src.reshape(E_PAD // B_CONV, B_CONV)
    dst_c = dst.reshape(E_PAD // B_CONV, B_CONV)
    src_p = src.reshape(E_PAD // B_PAIR, B_PAIR)
    dst_p = dst.reshape(E_PAD // B_PAIR, B_PAIR)

    h, g = _node_embed(at, rf, pf, p["atom_emb"], p["W_feat"], p["Wm0"])
    sumsq = _pos_sumsq(px, py, pz, src, dst).reshape(E_PAD, 1)
    ea, ep0 = _edge_base(sumsq, bt, p)
    ep1, ep2 = _edge_ep12(ea, p)

    eye = jnp.eye(HID, dtype=jnp.float32)
    for i, ep in enumerate((ep0, ep1, ep2)):
        agg = _conv_pass(g, ep, src_c, dst_c)
        wnext = p["Wm%d" % (i + 1)] if i < 2 else eye
        h, g = _node_update(h, agg, p["Wu%d" % i], p["bu%d" % i], wnext)

    prod = _pair_pass(g, src_p, dst_p)
    edge_inv, el = _head(prod, ea, sumsq, p)

    return edge_inv, bond_index, el
